# Initial kernel scaffold; baseline (speedup 1.0000x reference)
import jax
import jax.numpy as jnp
from jax.experimental import pallas as pl


def kernel(x, node_depth, edge_index, edge_attr, batch, emb_type, emb_attr, emb_depth, We, be, W1, b1, W2, b2, eps, Wp, bp):
    raise NotImplementedError("write your pallas kernel here")


# speedup vs baseline: 1.8485x; 1.8485x over previous
import jax
import jax.numpy as jnp
from jax import lax
from jax.experimental import pallas as pl
from jax.experimental.pallas import tpu as pltpu
from jax.experimental.pallas import tpu_sc as plsc

N = 10000
E = 320000
D = 128
DE = 16
NLAYER = 4
G = 128
S = 5
V = 5002

NC = 2
NS = 16
NW = NC * NS

NPAD = 10240
NROWS = NPAD // 128
EPAD = 327680
ER = EPAD // 128
RPW = ER // NW
CR = 1
CE = CR * 128
NCHUNK = RPW // CR
SROW = NPAD // NS

VPAD = 5120
VBLK = 640

_mesh = plsc.VectorSubcoreMesh(core_axis_name="c", subcore_axis_name="s")


def _enc_body(x0_hbm, x1_hbm, dd_hbm, ttype, tattr, tdepth, h0_hbm,
              ix0, ix1, ixd, b0, b1, b2, sem):
    c = lax.axis_index("c")
    s = lax.axis_index("s")
    w = s * NC + c
    start = jnp.where(w < 16, 3 * w, 48 + 2 * (w - 16))
    count = jnp.where(w < 16, 3, 2)

    def do_row(j, carry):
        row = start + j
        pltpu.sync_copy(x0_hbm.at[row], ix0.at[0])
        pltpu.sync_copy(x1_hbm.at[row], ix1.at[0])
        pltpu.sync_copy(dd_hbm.at[row], ixd.at[0])
        cp0 = pltpu.async_copy(ttype.at[ix0.at[0]], b0, sem)
        cp0.wait()
        cp1 = pltpu.async_copy(tattr.at[ix1.at[0]], b1, sem)
        cp1.wait()
        cp2 = pltpu.async_copy(tdepth.at[ixd.at[0]], b2, sem)
        cp2.wait()

        def srow(r, carry2):
            for jj in range(8):
                dsl = pl.ds(jj * 16, 16)
                b0[r, dsl] = b0[r, dsl] + b1[r, dsl] + b2[r, dsl]
            return carry2

        lax.fori_loop(0, 128, srow, 0)
        pltpu.sync_copy(b0, h0_hbm.at[pl.ds(row * 128, 128)])
        return carry

    lax.fori_loop(0, count, do_row, 0)


def _node_encoder(x0r, x1r, ddr, emb_type, emb_attr, emb_depth):
    f = pl.kernel(
        _enc_body,
        out_type=jax.ShapeDtypeStruct((NPAD, D), jnp.float32),
        mesh=_mesh,
        scratch_types=[
            pltpu.VMEM((1, 128), jnp.int32),
            pltpu.VMEM((1, 128), jnp.int32),
            pltpu.VMEM((1, 128), jnp.int32),
            pltpu.VMEM((128, D), jnp.float32),
            pltpu.VMEM((128, D), jnp.float32),
            pltpu.VMEM((128, D), jnp.float32),
            pltpu.SemaphoreType.DMA,
        ],
    )
    return f(x0r, x1r, ddr, emb_type, emb_attr, emb_depth)


def _edge_body(src_hbm, dst_hbm, e_hbm, h_hbm, aggp_hbm,
               sidx, didx, ebuf, hbuf, acc, sem):
    c = lax.axis_index("c")
    s = lax.axis_index("s")
    w = s * NC + c

    def zrow(r, carry):
        for jj in range(8):
            ebuf[r, pl.ds(jj * 16, 16)] = jnp.zeros((16,), jnp.float32)
        return carry

    lax.fori_loop(0, CE, zrow, 0)
    base = s * SROW
    for t in range(SROW // CE):
        pltpu.sync_copy(ebuf, acc.at[pl.ds(base + t * CE, CE)])
    plsc.subcore_barrier()

    def chunk(k, carry):
        row0 = w * RPW + k * CR
        pltpu.sync_copy(src_hbm.at[pl.ds(row0, CR)], sidx)
        pltpu.sync_copy(dst_hbm.at[pl.ds(row0, CR)], didx)
        pltpu.sync_copy(e_hbm.at[pl.ds(row0 * 128, CE)], ebuf)
        cps = [pltpu.async_copy(h_hbm.at[sidx.at[j]],
                                hbuf.at[pl.ds(j * 128, 128)], sem)
               for j in range(CR)]
        for cp in cps:
            cp.wait()

        def erow(r, carry2):
            for jj in range(8):
                dsl = pl.ds(jj * 16, 16)
                ebuf[r, dsl] = jnp.maximum(ebuf[r, dsl] + hbuf[r, dsl], 0.0)
            return carry2

        lax.fori_loop(0, CE, erow, 0)
        for j in range(CR):
            pltpu.sync_copy(ebuf.at[pl.ds(j * 128, 128)],
                            acc.at[didx.at[j]], add=True)
        return carry

    lax.fori_loop(0, NCHUNK, chunk, 0)
    plsc.subcore_barrier()
    pltpu.sync_copy(acc.at[pl.ds(base, SROW)],
                    aggp_hbm.at[c, pl.ds(base, SROW)])


def _edge_aggregate(src2d, dst2d, e_l, h):
    f = pl.kernel(
        _edge_body,
        out_type=jax.ShapeDtypeStruct((NC, NPAD, D), jnp.float32),
        mesh=_mesh,
        scratch_types=[
            pltpu.VMEM((CR, 128), jnp.int32),
            pltpu.VMEM((CR, 128), jnp.int32),
            pltpu.VMEM((CE, D), jnp.float32),
            pltpu.VMEM((CE, D), jnp.float32),
            pltpu.VMEM_SHARED((NPAD, D), jnp.float32),
            pltpu.SemaphoreType.DMA,
        ],
    )
    return f(src2d, dst2d, e_l, h)


EBLK = 2048


def _ee_body(ea_ref, we_ref, be_ref, o0, o1, o2, o3):
    a = ea_ref[...]
    outs = (o0, o1, o2, o3)
    for l in range(NLAYER):
        outs[l][...] = (jnp.dot(a, we_ref[l],
                                preferred_element_type=jnp.float32)
                        + be_ref[l:l + 1, :])


def _edge_encoder(ea_pad, We, be):
    nsteps = EPAD // EBLK
    return pl.pallas_call(
        _ee_body,
        grid=(nsteps,),
        in_specs=[
            pl.BlockSpec((EBLK, DE), lambda i: (i, 0)),
            pl.BlockSpec((NLAYER, DE, D), lambda i: (0, 0, 0)),
            pl.BlockSpec((NLAYER, D), lambda i: (0, 0)),
        ],
        out_specs=tuple(pl.BlockSpec((EBLK, D), lambda i: (i, 0))
                        for _ in range(NLAYER)),
        out_shape=tuple(jax.ShapeDtypeStruct((EPAD, D), jnp.float32)
                        for _ in range(NLAYER)),
        compiler_params=pltpu.CompilerParams(
            dimension_semantics=("arbitrary",)),
    )(ea_pad, We, be)


MBLK = 1024


def _mlp_body(h_ref, a0_ref, a1_ref, ep_ref, w1_ref, b1_ref, w2_ref, b2_ref,
              o_ref):
    t = h_ref[...] * ep_ref[...] + a0_ref[...] + a1_ref[...]
    z = jnp.maximum(
        jnp.dot(t, w1_ref[...], preferred_element_type=jnp.float32)
        + b1_ref[...], 0.0)
    o_ref[...] = jnp.maximum(
        jnp.dot(z, w2_ref[...], preferred_element_type=jnp.float32)
        + b2_ref[...], 0.0)


def _mlp(h, a0, a1, epsv, W1l, b1l, W2l, b2l):
    nsteps = NPAD // MBLK
    return pl.pallas_call(
        _mlp_body,
        grid=(nsteps,),
        in_specs=[
            pl.BlockSpec((MBLK, D), lambda i: (i, 0)),
            pl.BlockSpec((MBLK, D), lambda i: (i, 0)),
            pl.BlockSpec((MBLK, D), lambda i: (i, 0)),
            pl.BlockSpec((1, D), lambda i: (0, 0)),
            pl.BlockSpec((D, D), lambda i: (0, 0)),
            pl.BlockSpec((1, D), lambda i: (0, 0)),
            pl.BlockSpec((D, D), lambda i: (0, 0)),
            pl.BlockSpec((1, D), lambda i: (0, 0)),
        ],
        out_specs=pl.BlockSpec((MBLK, D), lambda i: (i, 0)),
        out_shape=jax.ShapeDtypeStruct((NPAD, D), jnp.float32),
        compiler_params=pltpu.CompilerParams(
            dimension_semantics=("arbitrary",)),
    )(h, a0, a1, epsv, W1l, b1l, W2l, b2l)


PBLK = 1024


def _pool_body(h_ref, b_ref, o_ref, acc_s, acc_c):
    i = pl.program_id(0)

    @pl.when(i == 0)
    def _init():
        acc_s[...] = jnp.zeros_like(acc_s)
        acc_c[...] = jnp.zeros_like(acc_c)

    bt = b_ref[...].reshape(1, PBLK)
    gi = lax.broadcasted_iota(jnp.int32, (G, PBLK), 0)
    oh = (gi == bt).astype(jnp.float32)
    acc_s[...] += jnp.dot(oh, h_ref[...], preferred_element_type=jnp.float32)
    acc_c[...] = acc_c[...] + jnp.sum(oh, axis=1, keepdims=True)

    @pl.when(i == NPAD // PBLK - 1)
    def _fin():
        o_ref[...] = acc_s[...] / jnp.maximum(acc_c[...], 1.0)


def _pool(h, batch2d):
    nsteps = NPAD // PBLK
    return pl.pallas_call(
        _pool_body,
        grid=(nsteps,),
        in_specs=[
            pl.BlockSpec((PBLK, D), lambda i: (i, 0)),
            pl.BlockSpec((PBLK, 1), lambda i: (i, 0)),
        ],
        out_specs=pl.BlockSpec((G, D), lambda i: (0, 0)),
        out_shape=jax.ShapeDtypeStruct((G, D), jnp.float32),
        scratch_shapes=[
            pltpu.VMEM((G, D), jnp.float32),
            pltpu.VMEM((G, 128), jnp.float32),
        ],
        compiler_params=pltpu.CompilerParams(
            dimension_semantics=("arbitrary",)),
    )(h, batch2d)


def _head_body(hg_ref, wp_ref, bp_ref, o_ref):
    o_ref[0] = (jnp.dot(hg_ref[...], wp_ref[0],
                        preferred_element_type=jnp.float32)
                + bp_ref[0])


def _heads(hg, Wp_pad, bp_pad):
    return pl.pallas_call(
        _head_body,
        grid=(S, VPAD // VBLK),
        in_specs=[
            pl.BlockSpec((G, D), lambda s, v: (0, 0)),
            pl.BlockSpec((1, D, VBLK), lambda s, v: (s, 0, v)),
            pl.BlockSpec((1, 1, VBLK), lambda s, v: (s, 0, v)),
        ],
        out_specs=pl.BlockSpec((1, G, VBLK), lambda s, v: (s, 0, v)),
        out_shape=jax.ShapeDtypeStruct((S, G, VPAD), jnp.float32),
        compiler_params=pltpu.CompilerParams(
            dimension_semantics=("arbitrary", "arbitrary")),
    )(hg, Wp_pad, bp_pad)


def kernel(x, node_depth, edge_index, edge_attr, batch,
           emb_type, emb_attr, emb_depth,
           We, be, W1, b1, W2, b2, eps, Wp, bp):
    f32 = jnp.float32
    i32 = jnp.int32

    x0r = jnp.pad(x[:, 0].astype(i32), (0, NPAD - N)).reshape(NROWS, 128)
    x1r = jnp.pad(x[:, 1].astype(i32), (0, NPAD - N)).reshape(NROWS, 128)
    ddr = jnp.pad(node_depth[:, 0].astype(i32),
                  (0, NPAD - N)).reshape(NROWS, 128)
    src2d = jnp.pad(edge_index[0].astype(i32),
                    (0, EPAD - E)).reshape(ER, 128)
    dst2d = jnp.pad(edge_index[1].astype(i32), (0, EPAD - E),
                    constant_values=N).reshape(ER, 128)
    ea_pad = jnp.pad(edge_attr.astype(f32), ((0, EPAD - E), (0, 0)))
    batch2d = jnp.pad(batch.astype(i32), (0, NPAD - N),
                      constant_values=G).reshape(NPAD, 1)

    h = _node_encoder(x0r, x1r, ddr, emb_type.astype(f32),
                      emb_attr.astype(f32), emb_depth.astype(f32))
    e_list = _edge_encoder(ea_pad, We.astype(f32), be.astype(f32))

    for l in range(NLAYER):
        aggp = _edge_aggregate(src2d, dst2d, e_list[l], h)
        epsv = (1.0 + eps[l]) * jnp.ones((1, D), f32)
        h = _mlp(h, aggp[0], aggp[1], epsv,
                 W1[l].astype(f32), b1[l][None, :].astype(f32),
                 W2[l].astype(f32), b2[l][None, :].astype(f32))

    hg = _pool(h, batch2d)
    Wp_pad = jnp.pad(Wp.astype(f32), ((0, 0), (0, 0), (0, VPAD - V)))
    bp_pad = jnp.pad(bp.astype(f32), ((0, 0), (0, VPAD - V))).reshape(S, 1, VPAD)
    preds = _heads(hg, Wp_pad, bp_pad)
    return preds[:, :, :V]


# speedup vs baseline: 1.9646x; 1.0628x over previous
import jax
import jax.numpy as jnp
from jax import lax
from jax.experimental import pallas as pl
from jax.experimental.pallas import tpu as pltpu
from jax.experimental.pallas import tpu_sc as plsc

N = 10000
E = 320000
D = 128
DE = 16
NLAYER = 4
G = 128
S = 5
V = 5002

NC = 2
NS = 16
NW = NC * NS

NPAD = 10240
NROWS = NPAD // 128
EPAD = 327680
ER = EPAD // 128
RPW = ER // NW
CE = 64
ECHUNKS = EPAD // CE
CPW = ECHUNKS // NW
NBODY = CPW // 2
SROW = NPAD // NS

VPAD = 5120
VBLK = 640

_mesh = plsc.VectorSubcoreMesh(core_axis_name="c", subcore_axis_name="s")


def _enc_body(x0_hbm, x1_hbm, dd_hbm, ttype, tattr, tdepth, h0_hbm,
              ix0, ix1, ixd, b0, b1, b2, sem):
    c = lax.axis_index("c")
    s = lax.axis_index("s")
    w = s * NC + c
    start = jnp.where(w < 16, 3 * w, 48 + 2 * (w - 16))
    count = jnp.where(w < 16, 3, 2)

    def do_row(j, carry):
        row = start + j
        pltpu.sync_copy(x0_hbm.at[row], ix0.at[0])
        pltpu.sync_copy(x1_hbm.at[row], ix1.at[0])
        pltpu.sync_copy(dd_hbm.at[row], ixd.at[0])
        cp0 = pltpu.async_copy(ttype.at[ix0.at[0]], b0, sem)
        cp0.wait()
        cp1 = pltpu.async_copy(tattr.at[ix1.at[0]], b1, sem)
        cp1.wait()
        cp2 = pltpu.async_copy(tdepth.at[ixd.at[0]], b2, sem)
        cp2.wait()

        def srow(r, carry2):
            for jj in range(8):
                dsl = pl.ds(jj * 16, 16)
                b0[r, dsl] = b0[r, dsl] + b1[r, dsl] + b2[r, dsl]
            return carry2

        lax.fori_loop(0, 128, srow, 0)
        pltpu.sync_copy(b0, h0_hbm.at[pl.ds(row * 128, 128)])
        return carry

    lax.fori_loop(0, count, do_row, 0)


def _node_encoder(x0r, x1r, ddr, emb_type, emb_attr, emb_depth):
    f = pl.kernel(
        _enc_body,
        out_type=jax.ShapeDtypeStruct((NPAD, D), jnp.float32),
        mesh=_mesh,
        scratch_types=[
            pltpu.VMEM((1, 128), jnp.int32),
            pltpu.VMEM((1, 128), jnp.int32),
            pltpu.VMEM((1, 128), jnp.int32),
            pltpu.VMEM((128, D), jnp.float32),
            pltpu.VMEM((128, D), jnp.float32),
            pltpu.VMEM((128, D), jnp.float32),
            pltpu.SemaphoreType.DMA,
        ],
    )
    return f(x0r, x1r, ddr, emb_type, emb_attr, emb_depth)


def _edge_body(src_hbm, dst_hbm, e_hbm, h_hbm, aggp_hbm,
               s0, s1, d0, d1, e0, e1, h0, h1, acc,
               sem_e, sem_g, sem_s):
    c = lax.axis_index("c")
    s = lax.axis_index("s")
    w = s * NC + c

    def zrow(r, carry):
        for jj in range(8):
            e0[r, pl.ds(jj * 16, 16)] = jnp.zeros((16,), jnp.float32)
        return carry

    lax.fori_loop(0, CE, zrow, 0)
    base = s * SROW
    for t in range(SROW // CE):
        pltpu.sync_copy(e0, acc.at[pl.ds(base + t * CE, CE)])
    plsc.subcore_barrier()

    sb = (s0, s1)
    db = (d0, d1)
    eb = (e0, e1)
    hb = (h0, h1)

    def compute(ebuf, hbuf):
        def erow(r, carry2):
            for jj in range(8):
                dsl = pl.ds(jj * 16, 16)
                ebuf[r, dsl] = jnp.maximum(ebuf[r, dsl] + hbuf[r, dsl], 0.0)
            return carry2

        lax.fori_loop(0, CE, erow, 0)

    def body(k, carry):
        c0 = w * CPW + k * 2
        cps = []
        for b in range(2):
            ck = c0 + b
            pltpu.sync_copy(src_hbm.at[pl.ds(ck, 1)], sb[b])
            pltpu.sync_copy(dst_hbm.at[pl.ds(ck, 1)], db[b])
            cps.append(pltpu.async_copy(e_hbm.at[pl.ds(ck * CE, CE)],
                                        eb[b], sem_e))
            cps.append(pltpu.async_copy(h_hbm.at[sb[b].at[0]], hb[b], sem_g))
        scs = []
        for b in range(2):
            cps[2 * b].wait()
            cps[2 * b + 1].wait()
            compute(eb[b], hb[b])
            scs.append(pltpu.async_copy(eb[b], acc.at[db[b].at[0]], sem_s,
                                        add=True))
        for sc in scs:
            sc.wait()
        return carry

    lax.fori_loop(0, NBODY, body, 0)
    plsc.subcore_barrier()
    pltpu.sync_copy(acc.at[pl.ds(base, SROW)],
                    aggp_hbm.at[c, pl.ds(base, SROW)])


def _edge_aggregate(src2d, dst2d, e_l, h):
    f = pl.kernel(
        _edge_body,
        out_type=jax.ShapeDtypeStruct((NC, NPAD, D), jnp.float32),
        mesh=_mesh,
        scratch_types=[
            pltpu.VMEM((1, CE), jnp.int32),
            pltpu.VMEM((1, CE), jnp.int32),
            pltpu.VMEM((1, CE), jnp.int32),
            pltpu.VMEM((1, CE), jnp.int32),
            pltpu.VMEM((CE, D), jnp.float32),
            pltpu.VMEM((CE, D), jnp.float32),
            pltpu.VMEM((CE, D), jnp.float32),
            pltpu.VMEM((CE, D), jnp.float32),
            pltpu.VMEM_SHARED((NPAD, D), jnp.float32),
            pltpu.SemaphoreType.DMA,
            pltpu.SemaphoreType.DMA,
            pltpu.SemaphoreType.DMA,
        ],
    )
    return f(src2d, dst2d, e_l, h)


EBLK = 2048


def _ee_body(ea_ref, we_ref, be_ref, o0, o1, o2, o3):
    a = ea_ref[...]
    outs = (o0, o1, o2, o3)
    for l in range(NLAYER):
        outs[l][...] = (jnp.dot(a, we_ref[l],
                                preferred_element_type=jnp.float32)
                        + be_ref[l:l + 1, :])


def _edge_encoder(ea_pad, We, be):
    nsteps = EPAD // EBLK
    return pl.pallas_call(
        _ee_body,
        grid=(nsteps,),
        in_specs=[
            pl.BlockSpec((EBLK, DE), lambda i: (i, 0)),
            pl.BlockSpec((NLAYER, DE, D), lambda i: (0, 0, 0)),
            pl.BlockSpec((NLAYER, D), lambda i: (0, 0)),
        ],
        out_specs=tuple(pl.BlockSpec((EBLK, D), lambda i: (i, 0))
                        for _ in range(NLAYER)),
        out_shape=tuple(jax.ShapeDtypeStruct((EPAD, D), jnp.float32)
                        for _ in range(NLAYER)),
        compiler_params=pltpu.CompilerParams(
            dimension_semantics=("arbitrary",)),
    )(ea_pad, We, be)


MBLK = 1024


def _mlp_body(h_ref, a0_ref, a1_ref, ep_ref, w1_ref, b1_ref, w2_ref, b2_ref,
              o_ref):
    t = h_ref[...] * ep_ref[...] + a0_ref[...] + a1_ref[...]
    z = jnp.maximum(
        jnp.dot(t, w1_ref[...], preferred_element_type=jnp.float32)
        + b1_ref[...], 0.0)
    o_ref[...] = jnp.maximum(
        jnp.dot(z, w2_ref[...], preferred_element_type=jnp.float32)
        + b2_ref[...], 0.0)


def _mlp(h, a0, a1, epsv, W1l, b1l, W2l, b2l):
    nsteps = NPAD // MBLK
    return pl.pallas_call(
        _mlp_body,
        grid=(nsteps,),
        in_specs=[
            pl.BlockSpec((MBLK, D), lambda i: (i, 0)),
            pl.BlockSpec((MBLK, D), lambda i: (i, 0)),
            pl.BlockSpec((MBLK, D), lambda i: (i, 0)),
            pl.BlockSpec((1, D), lambda i: (0, 0)),
            pl.BlockSpec((D, D), lambda i: (0, 0)),
            pl.BlockSpec((1, D), lambda i: (0, 0)),
            pl.BlockSpec((D, D), lambda i: (0, 0)),
            pl.BlockSpec((1, D), lambda i: (0, 0)),
        ],
        out_specs=pl.BlockSpec((MBLK, D), lambda i: (i, 0)),
        out_shape=jax.ShapeDtypeStruct((NPAD, D), jnp.float32),
        compiler_params=pltpu.CompilerParams(
            dimension_semantics=("arbitrary",)),
    )(h, a0, a1, epsv, W1l, b1l, W2l, b2l)


PBLK = 1024


def _pool_body(h_ref, b_ref, o_ref, acc_s, acc_c):
    i = pl.program_id(0)

    @pl.when(i == 0)
    def _init():
        acc_s[...] = jnp.zeros_like(acc_s)
        acc_c[...] = jnp.zeros_like(acc_c)

    bt = b_ref[...].reshape(1, PBLK)
    gi = lax.broadcasted_iota(jnp.int32, (G, PBLK), 0)
    oh = (gi == bt).astype(jnp.float32)
    acc_s[...] += jnp.dot(oh, h_ref[...], preferred_element_type=jnp.float32)
    acc_c[...] = acc_c[...] + jnp.sum(oh, axis=1, keepdims=True)

    @pl.when(i == NPAD // PBLK - 1)
    def _fin():
        o_ref[...] = acc_s[...] / jnp.maximum(acc_c[...], 1.0)


def _pool(h, batch2d):
    nsteps = NPAD // PBLK
    return pl.pallas_call(
        _pool_body,
        grid=(nsteps,),
        in_specs=[
            pl.BlockSpec((PBLK, D), lambda i: (i, 0)),
            pl.BlockSpec((PBLK, 1), lambda i: (i, 0)),
        ],
        out_specs=pl.BlockSpec((G, D), lambda i: (0, 0)),
        out_shape=jax.ShapeDtypeStruct((G, D), jnp.float32),
        scratch_shapes=[
            pltpu.VMEM((G, D), jnp.float32),
            pltpu.VMEM((G, 128), jnp.float32),
        ],
        compiler_params=pltpu.CompilerParams(
            dimension_semantics=("arbitrary",)),
    )(h, batch2d)


def _head_body(hg_ref, wp_ref, bp_ref, o_ref):
    o_ref[0] = (jnp.dot(hg_ref[...], wp_ref[0],
                        preferred_element_type=jnp.float32)
                + bp_ref[0])


def _heads(hg, Wp_pad, bp_pad):
    return pl.pallas_call(
        _head_body,
        grid=(S, VPAD // VBLK),
        in_specs=[
            pl.BlockSpec((G, D), lambda s, v: (0, 0)),
            pl.BlockSpec((1, D, VBLK), lambda s, v: (s, 0, v)),
            pl.BlockSpec((1, 1, VBLK), lambda s, v: (s, 0, v)),
        ],
        out_specs=pl.BlockSpec((1, G, VBLK), lambda s, v: (s, 0, v)),
        out_shape=jax.ShapeDtypeStruct((S, G, VPAD), jnp.float32),
        compiler_params=pltpu.CompilerParams(
            dimension_semantics=("arbitrary", "arbitrary")),
    )(hg, Wp_pad, bp_pad)


def kernel(x, node_depth, edge_index, edge_attr, batch,
           emb_type, emb_attr, emb_depth,
           We, be, W1, b1, W2, b2, eps, Wp, bp):
    f32 = jnp.float32
    i32 = jnp.int32

    x0r = jnp.pad(x[:, 0].astype(i32), (0, NPAD - N)).reshape(NROWS, 128)
    x1r = jnp.pad(x[:, 1].astype(i32), (0, NPAD - N)).reshape(NROWS, 128)
    ddr = jnp.pad(node_depth[:, 0].astype(i32),
                  (0, NPAD - N)).reshape(NROWS, 128)
    src2d = jnp.pad(edge_index[0].astype(i32),
                    (0, EPAD - E)).reshape(ECHUNKS, CE)
    dst2d = jnp.pad(edge_index[1].astype(i32), (0, EPAD - E),
                    constant_values=N).reshape(ECHUNKS, CE)
    ea_pad = jnp.pad(edge_attr.astype(f32), ((0, EPAD - E), (0, 0)))
    batch2d = jnp.pad(batch.astype(i32), (0, NPAD - N),
                      constant_values=G).reshape(NPAD, 1)

    h = _node_encoder(x0r, x1r, ddr, emb_type.astype(f32),
                      emb_attr.astype(f32), emb_depth.astype(f32))
    e_list = _edge_encoder(ea_pad, We.astype(f32), be.astype(f32))

    for l in range(NLAYER):
        aggp = _edge_aggregate(src2d, dst2d, e_list[l], h)
        epsv = (1.0 + eps[l]) * jnp.ones((1, D), f32)
        h = _mlp(h, aggp[0], aggp[1], epsv,
                 W1[l].astype(f32), b1[l][None, :].astype(f32),
                 W2[l].astype(f32), b2[l][None, :].astype(f32))

    hg = _pool(h, batch2d)
    Wp_pad = jnp.pad(Wp.astype(f32), ((0, 0), (0, 0), (0, VPAD - V)))
    bp_pad = jnp.pad(bp.astype(f32), ((0, 0), (0, VPAD - V))).reshape(S, 1, VPAD)
    preds = _heads(hg, Wp_pad, bp_pad)
    return preds[:, :, :V]


# speedup vs baseline: 2.1084x; 1.0732x over previous
import jax
import jax.numpy as jnp
from jax import lax
from jax.experimental import pallas as pl
from jax.experimental.pallas import tpu as pltpu
from jax.experimental.pallas import tpu_sc as plsc

N = 10000
E = 320000
D = 128
DE = 16
NLAYER = 4
G = 128
S = 5
V = 5002

NC = 2
NS = 16
NW = NC * NS

NPAD = 10240
NROWS = NPAD // 128
EPAD = 327680
ER = EPAD // 128
RPW = ER // NW
CE = 64
ECHUNKS = EPAD // CE
CPW = ECHUNKS // NW
CBLK = 32
SROW = NPAD // NS

VPAD = 5120
VBLK = 640

_mesh = plsc.VectorSubcoreMesh(core_axis_name="c", subcore_axis_name="s")


def _enc_body(x0_hbm, x1_hbm, dd_hbm, ttype, tattr, tdepth, h0_hbm,
              ix0, ix1, ixd, b0, b1, b2, sem):
    c = lax.axis_index("c")
    s = lax.axis_index("s")
    w = s * NC + c
    start = jnp.where(w < 16, 3 * w, 48 + 2 * (w - 16))
    count = jnp.where(w < 16, 3, 2)

    def do_row(j, carry):
        row = start + j
        pltpu.sync_copy(x0_hbm.at[row], ix0.at[0])
        pltpu.sync_copy(x1_hbm.at[row], ix1.at[0])
        pltpu.sync_copy(dd_hbm.at[row], ixd.at[0])
        cp0 = pltpu.async_copy(ttype.at[ix0.at[0]], b0, sem)
        cp0.wait()
        cp1 = pltpu.async_copy(tattr.at[ix1.at[0]], b1, sem)
        cp1.wait()
        cp2 = pltpu.async_copy(tdepth.at[ixd.at[0]], b2, sem)
        cp2.wait()

        def srow(r, carry2):
            for jj in range(8):
                dsl = pl.ds(jj * 16, 16)
                b0[r, dsl] = b0[r, dsl] + b1[r, dsl] + b2[r, dsl]
            return carry2

        lax.fori_loop(0, 128, srow, 0)
        pltpu.sync_copy(b0, h0_hbm.at[pl.ds(row * 128, 128)])
        return carry

    lax.fori_loop(0, count, do_row, 0)


def _node_encoder(x0r, x1r, ddr, emb_type, emb_attr, emb_depth):
    f = pl.kernel(
        _enc_body,
        out_type=jax.ShapeDtypeStruct((NPAD, D), jnp.float32),
        mesh=_mesh,
        scratch_types=[
            pltpu.VMEM((1, 128), jnp.int32),
            pltpu.VMEM((1, 128), jnp.int32),
            pltpu.VMEM((1, 128), jnp.int32),
            pltpu.VMEM((128, D), jnp.float32),
            pltpu.VMEM((128, D), jnp.float32),
            pltpu.VMEM((128, D), jnp.float32),
            pltpu.SemaphoreType.DMA,
        ],
    )
    return f(x0r, x1r, ddr, emb_type, emb_attr, emb_depth)


def _edge_body(src_hbm, dst_hbm, e_hbm, h_hbm, aggp_hbm,
               s0, d0, e0, e1, h0, h1, acc,
               sem_e, sem_g, sem_s):
    c = lax.axis_index("c")
    s = lax.axis_index("s")
    w = s * NC + c

    def zrow(r, carry):
        for jj in range(8):
            e0[r, pl.ds(jj * 16, 16)] = jnp.zeros((16,), jnp.float32)
        return carry

    lax.fori_loop(0, CE, zrow, 0)
    base = s * SROW
    for t in range(SROW // CE):
        pltpu.sync_copy(e0, acc.at[pl.ds(base + t * CE, CE)])
    plsc.subcore_barrier()

    eb = (e0, e1)
    hb = (h0, h1)

    def compute(ebuf, hbuf):
        def erow(r, carry2):
            for jj in range(8):
                dsl = pl.ds(jj * 16, 16)
                ebuf[r, dsl] = jnp.maximum(ebuf[r, dsl] + hbuf[r, dsl], 0.0)
            return carry2

        lax.fori_loop(0, CE, erow, 0)

    def block(blk, carry):
        b0 = w * CPW + blk * CBLK
        pltpu.sync_copy(src_hbm.at[pl.ds(b0, CBLK)], s0)
        pltpu.sync_copy(dst_hbm.at[pl.ds(b0, CBLK)], d0)

        def body(k, carry2):
            cps = []
            for b in range(2):
                j = k * 2 + b
                cps.append(pltpu.async_copy(
                    e_hbm.at[pl.ds((b0 + j) * CE, CE)], eb[b], sem_e))
                cps.append(pltpu.async_copy(h_hbm.at[s0.at[j]], hb[b], sem_g))
            scs = []
            for b in range(2):
                j = k * 2 + b
                cps[2 * b].wait()
                cps[2 * b + 1].wait()
                compute(eb[b], hb[b])
                scs.append(pltpu.async_copy(eb[b], acc.at[d0.at[j]], sem_s,
                                            add=True))
            for sc in scs:
                sc.wait()
            return carry2

        lax.fori_loop(0, CBLK // 2, body, 0)
        return carry

    lax.fori_loop(0, CPW // CBLK, block, 0)
    plsc.subcore_barrier()
    pltpu.sync_copy(acc.at[pl.ds(base, SROW)],
                    aggp_hbm.at[c, pl.ds(base, SROW)])


def _edge_aggregate(src2d, dst2d, e_l, h):
    f = pl.kernel(
        _edge_body,
        out_type=jax.ShapeDtypeStruct((NC, NPAD, D), jnp.float32),
        mesh=_mesh,
        scratch_types=[
            pltpu.VMEM((CBLK, CE), jnp.int32),
            pltpu.VMEM((CBLK, CE), jnp.int32),
            pltpu.VMEM((CE, D), jnp.float32),
            pltpu.VMEM((CE, D), jnp.float32),
            pltpu.VMEM((CE, D), jnp.float32),
            pltpu.VMEM((CE, D), jnp.float32),
            pltpu.VMEM_SHARED((NPAD, D), jnp.float32),
            pltpu.SemaphoreType.DMA,
            pltpu.SemaphoreType.DMA,
            pltpu.SemaphoreType.DMA,
        ],
    )
    return f(src2d, dst2d, e_l, h)


EBLK = 2048


def _ee_body(ea_ref, we_ref, be_ref, o0, o1, o2, o3):
    a = ea_ref[...]
    outs = (o0, o1, o2, o3)
    for l in range(NLAYER):
        outs[l][...] = (jnp.dot(a, we_ref[l],
                                preferred_element_type=jnp.float32)
                        + be_ref[l:l + 1, :])


def _edge_encoder(ea_pad, We, be):
    nsteps = EPAD // EBLK
    return pl.pallas_call(
        _ee_body,
        grid=(nsteps,),
        in_specs=[
            pl.BlockSpec((EBLK, DE), lambda i: (i, 0)),
            pl.BlockSpec((NLAYER, DE, D), lambda i: (0, 0, 0)),
            pl.BlockSpec((NLAYER, D), lambda i: (0, 0)),
        ],
        out_specs=tuple(pl.BlockSpec((EBLK, D), lambda i: (i, 0))
                        for _ in range(NLAYER)),
        out_shape=tuple(jax.ShapeDtypeStruct((EPAD, D), jnp.float32)
                        for _ in range(NLAYER)),
        compiler_params=pltpu.CompilerParams(
            dimension_semantics=("arbitrary",)),
    )(ea_pad, We, be)


MBLK = 1024


def _mlp_body(h_ref, a0_ref, a1_ref, ep_ref, w1_ref, b1_ref, w2_ref, b2_ref,
              o_ref):
    t = h_ref[...] * ep_ref[...] + a0_ref[...] + a1_ref[...]
    z = jnp.maximum(
        jnp.dot(t, w1_ref[...], preferred_element_type=jnp.float32)
        + b1_ref[...], 0.0)
    o_ref[...] = jnp.maximum(
        jnp.dot(z, w2_ref[...], preferred_element_type=jnp.float32)
        + b2_ref[...], 0.0)


def _mlp(h, a0, a1, epsv, W1l, b1l, W2l, b2l):
    nsteps = NPAD // MBLK
    return pl.pallas_call(
        _mlp_body,
        grid=(nsteps,),
        in_specs=[
            pl.BlockSpec((MBLK, D), lambda i: (i, 0)),
            pl.BlockSpec((MBLK, D), lambda i: (i, 0)),
            pl.BlockSpec((MBLK, D), lambda i: (i, 0)),
            pl.BlockSpec((1, D), lambda i: (0, 0)),
            pl.BlockSpec((D, D), lambda i: (0, 0)),
            pl.BlockSpec((1, D), lambda i: (0, 0)),
            pl.BlockSpec((D, D), lambda i: (0, 0)),
            pl.BlockSpec((1, D), lambda i: (0, 0)),
        ],
        out_specs=pl.BlockSpec((MBLK, D), lambda i: (i, 0)),
        out_shape=jax.ShapeDtypeStruct((NPAD, D), jnp.float32),
        compiler_params=pltpu.CompilerParams(
            dimension_semantics=("arbitrary",)),
    )(h, a0, a1, epsv, W1l, b1l, W2l, b2l)


PBLK = 1024


def _pool_body(h_ref, b_ref, o_ref, acc_s, acc_c):
    i = pl.program_id(0)

    @pl.when(i == 0)
    def _init():
        acc_s[...] = jnp.zeros_like(acc_s)
        acc_c[...] = jnp.zeros_like(acc_c)

    bt = b_ref[...].reshape(1, PBLK)
    gi = lax.broadcasted_iota(jnp.int32, (G, PBLK), 0)
    oh = (gi == bt).astype(jnp.float32)
    acc_s[...] += jnp.dot(oh, h_ref[...], preferred_element_type=jnp.float32)
    acc_c[...] = acc_c[...] + jnp.sum(oh, axis=1, keepdims=True)

    @pl.when(i == NPAD // PBLK - 1)
    def _fin():
        o_ref[...] = acc_s[...] / jnp.maximum(acc_c[...], 1.0)


def _pool(h, batch2d):
    nsteps = NPAD // PBLK
    return pl.pallas_call(
        _pool_body,
        grid=(nsteps,),
        in_specs=[
            pl.BlockSpec((PBLK, D), lambda i: (i, 0)),
            pl.BlockSpec((PBLK, 1), lambda i: (i, 0)),
        ],
        out_specs=pl.BlockSpec((G, D), lambda i: (0, 0)),
        out_shape=jax.ShapeDtypeStruct((G, D), jnp.float32),
        scratch_shapes=[
            pltpu.VMEM((G, D), jnp.float32),
            pltpu.VMEM((G, 128), jnp.float32),
        ],
        compiler_params=pltpu.CompilerParams(
            dimension_semantics=("arbitrary",)),
    )(h, batch2d)


def _head_body(hg_ref, wp_ref, bp_ref, o_ref):
    o_ref[0] = (jnp.dot(hg_ref[...], wp_ref[0],
                        preferred_element_type=jnp.float32)
                + bp_ref[0])


def _heads(hg, Wp_pad, bp_pad):
    return pl.pallas_call(
        _head_body,
        grid=(S, VPAD // VBLK),
        in_specs=[
            pl.BlockSpec((G, D), lambda s, v: (0, 0)),
            pl.BlockSpec((1, D, VBLK), lambda s, v: (s, 0, v)),
            pl.BlockSpec((1, 1, VBLK), lambda s, v: (s, 0, v)),
        ],
        out_specs=pl.BlockSpec((1, G, VBLK), lambda s, v: (s, 0, v)),
        out_shape=jax.ShapeDtypeStruct((S, G, VPAD), jnp.float32),
        compiler_params=pltpu.CompilerParams(
            dimension_semantics=("arbitrary", "arbitrary")),
    )(hg, Wp_pad, bp_pad)


def kernel(x, node_depth, edge_index, edge_attr, batch,
           emb_type, emb_attr, emb_depth,
           We, be, W1, b1, W2, b2, eps, Wp, bp):
    f32 = jnp.float32
    i32 = jnp.int32

    x0r = jnp.pad(x[:, 0].astype(i32), (0, NPAD - N)).reshape(NROWS, 128)
    x1r = jnp.pad(x[:, 1].astype(i32), (0, NPAD - N)).reshape(NROWS, 128)
    ddr = jnp.pad(node_depth[:, 0].astype(i32),
                  (0, NPAD - N)).reshape(NROWS, 128)
    src2d = jnp.pad(edge_index[0].astype(i32),
                    (0, EPAD - E)).reshape(ECHUNKS, CE)
    dst2d = jnp.pad(edge_index[1].astype(i32), (0, EPAD - E),
                    constant_values=N).reshape(ECHUNKS, CE)
    ea_pad = jnp.pad(edge_attr.astype(f32), ((0, EPAD - E), (0, 0)))
    batch2d = jnp.pad(batch.astype(i32), (0, NPAD - N),
                      constant_values=G).reshape(NPAD, 1)

    h = _node_encoder(x0r, x1r, ddr, emb_type.astype(f32),
                      emb_attr.astype(f32), emb_depth.astype(f32))
    e_list = _edge_encoder(ea_pad, We.astype(f32), be.astype(f32))

    for l in range(NLAYER):
        aggp = _edge_aggregate(src2d, dst2d, e_list[l], h)
        epsv = (1.0 + eps[l]) * jnp.ones((1, D), f32)
        h = _mlp(h, aggp[0], aggp[1], epsv,
                 W1[l].astype(f32), b1[l][None, :].astype(f32),
                 W2[l].astype(f32), b2[l][None, :].astype(f32))

    hg = _pool(h, batch2d)
    Wp_pad = jnp.pad(Wp.astype(f32), ((0, 0), (0, 0), (0, VPAD - V)))
    bp_pad = jnp.pad(bp.astype(f32), ((0, 0), (0, VPAD - V))).reshape(S, 1, VPAD)
    preds = _heads(hg, Wp_pad, bp_pad)
    return preds[:, :, :V]


# speedup vs baseline: 2.3212x; 1.1010x over previous
import jax
import jax.numpy as jnp
from jax import lax
from jax.experimental import pallas as pl
from jax.experimental.pallas import tpu as pltpu
from jax.experimental.pallas import tpu_sc as plsc

N = 10000
E = 320000
D = 128
DE = 16
NLAYER = 4
G = 128
S = 5
V = 5002

NC = 2
NS = 16
NW = NC * NS

NPAD = 10240
NROWS = NPAD // 128
EPAD = 327680
ER = EPAD // 128
RPW = ER // NW
CE = 64
ECHUNKS = EPAD // CE
CPW = ECHUNKS // NW
CBLK = 16
IB = CBLK + 8
NSUPER = CPW // (2 * CBLK)
SROW = NPAD // NS

VPAD = 5120
VBLK = 640

_mesh = plsc.VectorSubcoreMesh(core_axis_name="c", subcore_axis_name="s")


def _enc_body(x0_hbm, x1_hbm, dd_hbm, ttype, tattr, tdepth, h0_hbm,
              ix0, ix1, ixd, b0, b1, b2, sem):
    c = lax.axis_index("c")
    s = lax.axis_index("s")
    w = s * NC + c
    start = jnp.where(w < 16, 3 * w, 48 + 2 * (w - 16))
    count = jnp.where(w < 16, 3, 2)

    def do_row(j, carry):
        row = start + j
        pltpu.sync_copy(x0_hbm.at[row], ix0.at[0])
        pltpu.sync_copy(x1_hbm.at[row], ix1.at[0])
        pltpu.sync_copy(dd_hbm.at[row], ixd.at[0])
        cp0 = pltpu.async_copy(ttype.at[ix0.at[0]], b0, sem)
        cp0.wait()
        cp1 = pltpu.async_copy(tattr.at[ix1.at[0]], b1, sem)
        cp1.wait()
        cp2 = pltpu.async_copy(tdepth.at[ixd.at[0]], b2, sem)
        cp2.wait()

        def srow(r, carry2):
            for jj in range(8):
                dsl = pl.ds(jj * 16, 16)
                b0[r, dsl] = b0[r, dsl] + b1[r, dsl] + b2[r, dsl]
            return carry2

        lax.fori_loop(0, 128, srow, 0)
        pltpu.sync_copy(b0, h0_hbm.at[pl.ds(row * 128, 128)])
        return carry

    lax.fori_loop(0, count, do_row, 0)


def _node_encoder(x0r, x1r, ddr, emb_type, emb_attr, emb_depth):
    f = pl.kernel(
        _enc_body,
        out_type=jax.ShapeDtypeStruct((NPAD, D), jnp.float32),
        mesh=_mesh,
        scratch_types=[
            pltpu.VMEM((1, 128), jnp.int32),
            pltpu.VMEM((1, 128), jnp.int32),
            pltpu.VMEM((1, 128), jnp.int32),
            pltpu.VMEM((128, D), jnp.float32),
            pltpu.VMEM((128, D), jnp.float32),
            pltpu.VMEM((128, D), jnp.float32),
            pltpu.SemaphoreType.DMA,
        ],
    )
    return f(x0r, x1r, ddr, emb_type, emb_attr, emb_depth)


def _edge_body(src_hbm, dst_hbm, e_hbm, h_hbm, aggp_hbm,
               s0b, s1b, d0b, d1b, eb, hA, hB, ob, acc,
               sem_e, sem_g0, sem_g1, sem_s):
    c = lax.axis_index("c")
    s = lax.axis_index("s")
    w = s * NC + c
    w0 = w * CPW

    def zrow(r, carry):
        for jj in range(8):
            eb[r, pl.ds(jj * 16, 16)] = jnp.zeros((16,), jnp.float32)
        return carry

    lax.fori_loop(0, CE, zrow, 0)
    base = s * SROW
    for t in range(SROW // CE):
        pltpu.sync_copy(eb, acc.at[pl.ds(base + t * CE, CE)])
    plsc.subcore_barrier()

    sbufs = (s0b, s1b)
    dbufs = (d0b, d1b)

    def wait_bytes(dst, sem):
        pltpu.make_async_copy(e_hbm.at[pl.ds(0, CE)], dst, sem).wait()

    def compute(obuf, hbuf):
        def erow(r, carry2):
            for jj in range(8):
                dsl = pl.ds(jj * 16, 16)
                obuf[r, dsl] = jnp.maximum(eb[r, dsl] + hbuf[r, dsl], 0.0)
            return carry2

        lax.fori_loop(0, CE, erow, 0)

    pltpu.sync_copy(src_hbm.at[pl.ds(w0, IB)], s0b)
    pltpu.sync_copy(dst_hbm.at[pl.ds(w0, IB)], d0b)
    pltpu.async_copy(h_hbm.at[s0b.at[0]], hA, sem_g0)
    pltpu.async_copy(h_hbm.at[s0b.at[1]], hB, sem_g1)
    pltpu.async_copy(e_hbm.at[pl.ds(w0 * CE, CE)], eb, sem_e)

    def super_body(sp, carry):
        for par in range(2):
            sbuf = sbufs[par]
            dbuf = dbufs[par]
            bk = sp * 2 + par
            lb = bk * CBLK

            def load_idx():
                pltpu.sync_copy(src_hbm.at[pl.ds(w0 + lb, IB)], sbuf)
                pltpu.sync_copy(dst_hbm.at[pl.ds(w0 + lb, IB)], dbuf)

            if par == 0:
                pl.when(sp > 0)(load_idx)
            else:
                load_idx()

            def body(i2, carry2):
                for b, (hX, sg) in enumerate(((hA, sem_g0), (hB, sem_g1))):
                    jl = 2 * i2 + b
                    lc = lb + jl
                    gc = w0 + lc

                    wait_bytes(hX, sg)
                    wait_bytes(eb, sem_e)
                    pl.when(lc >= 1)(lambda: wait_bytes(ob, sem_s))
                    compute(ob, hX)
                    pltpu.async_copy(ob, acc.at[dbuf.at[jl]], sem_s, add=True)

                    def next_e(gc=gc):
                        pltpu.async_copy(
                            e_hbm.at[pl.ds((gc + 1) * CE, CE)], eb, sem_e)

                    pl.when(lc < CPW - 1)(next_e)

                    def next_g(jl=jl, hX=hX, sg=sg, sbuf=sbuf):
                        pltpu.async_copy(h_hbm.at[sbuf.at[jl + 2]], hX, sg)

                    pl.when(lc < CPW - 2)(next_g)
                return carry2

            lax.fori_loop(0, CBLK // 2, body, 0)
        return carry

    lax.fori_loop(0, NSUPER, super_body, 0)
    wait_bytes(ob, sem_s)
    plsc.subcore_barrier()
    pltpu.sync_copy(acc.at[pl.ds(base, SROW)],
                    aggp_hbm.at[c, pl.ds(base, SROW)])


def _edge_aggregate(src2d, dst2d, e_l, h):
    f = pl.kernel(
        _edge_body,
        out_type=jax.ShapeDtypeStruct((NC, NPAD, D), jnp.float32),
        mesh=_mesh,
        scratch_types=[
            pltpu.VMEM((IB, CE), jnp.int32),
            pltpu.VMEM((IB, CE), jnp.int32),
            pltpu.VMEM((IB, CE), jnp.int32),
            pltpu.VMEM((IB, CE), jnp.int32),
            pltpu.VMEM((CE, D), jnp.float32),
            pltpu.VMEM((CE, D), jnp.float32),
            pltpu.VMEM((CE, D), jnp.float32),
            pltpu.VMEM((CE, D), jnp.float32),
            pltpu.VMEM_SHARED((NPAD, D), jnp.float32),
            pltpu.SemaphoreType.DMA,
            pltpu.SemaphoreType.DMA,
            pltpu.SemaphoreType.DMA,
            pltpu.SemaphoreType.DMA,
        ],
    )
    return f(src2d, dst2d, e_l, h)


EBLK = 2048


def _ee_body(ea_ref, we_ref, be_ref, o0, o1, o2, o3):
    a = ea_ref[...]
    outs = (o0, o1, o2, o3)
    for l in range(NLAYER):
        outs[l][...] = (jnp.dot(a, we_ref[l],
                                preferred_element_type=jnp.float32)
                        + be_ref[l:l + 1, :])


def _edge_encoder(ea_pad, We, be):
    nsteps = EPAD // EBLK
    return pl.pallas_call(
        _ee_body,
        grid=(nsteps,),
        in_specs=[
            pl.BlockSpec((EBLK, DE), lambda i: (i, 0)),
            pl.BlockSpec((NLAYER, DE, D), lambda i: (0, 0, 0)),
            pl.BlockSpec((NLAYER, D), lambda i: (0, 0)),
        ],
        out_specs=tuple(pl.BlockSpec((EBLK, D), lambda i: (i, 0))
                        for _ in range(NLAYER)),
        out_shape=tuple(jax.ShapeDtypeStruct((EPAD, D), jnp.float32)
                        for _ in range(NLAYER)),
        compiler_params=pltpu.CompilerParams(
            dimension_semantics=("arbitrary",)),
    )(ea_pad, We, be)


MBLK = 1024


def _mlp_body(h_ref, a0_ref, a1_ref, ep_ref, w1_ref, b1_ref, w2_ref, b2_ref,
              o_ref):
    t = h_ref[...] * ep_ref[...] + a0_ref[...] + a1_ref[...]
    z = jnp.maximum(
        jnp.dot(t, w1_ref[...], preferred_element_type=jnp.float32)
        + b1_ref[...], 0.0)
    o_ref[...] = jnp.maximum(
        jnp.dot(z, w2_ref[...], preferred_element_type=jnp.float32)
        + b2_ref[...], 0.0)


def _mlp(h, a0, a1, epsv, W1l, b1l, W2l, b2l):
    nsteps = NPAD // MBLK
    return pl.pallas_call(
        _mlp_body,
        grid=(nsteps,),
        in_specs=[
            pl.BlockSpec((MBLK, D), lambda i: (i, 0)),
            pl.BlockSpec((MBLK, D), lambda i: (i, 0)),
            pl.BlockSpec((MBLK, D), lambda i: (i, 0)),
            pl.BlockSpec((1, D), lambda i: (0, 0)),
            pl.BlockSpec((D, D), lambda i: (0, 0)),
            pl.BlockSpec((1, D), lambda i: (0, 0)),
            pl.BlockSpec((D, D), lambda i: (0, 0)),
            pl.BlockSpec((1, D), lambda i: (0, 0)),
        ],
        out_specs=pl.BlockSpec((MBLK, D), lambda i: (i, 0)),
        out_shape=jax.ShapeDtypeStruct((NPAD, D), jnp.float32),
        compiler_params=pltpu.CompilerParams(
            dimension_semantics=("arbitrary",)),
    )(h, a0, a1, epsv, W1l, b1l, W2l, b2l)


PBLK = 1024


def _pool_body(h_ref, b_ref, o_ref, acc_s, acc_c):
    i = pl.program_id(0)

    @pl.when(i == 0)
    def _init():
        acc_s[...] = jnp.zeros_like(acc_s)
        acc_c[...] = jnp.zeros_like(acc_c)

    bt = b_ref[...].reshape(1, PBLK)
    gi = lax.broadcasted_iota(jnp.int32, (G, PBLK), 0)
    oh = (gi == bt).astype(jnp.float32)
    acc_s[...] += jnp.dot(oh, h_ref[...], preferred_element_type=jnp.float32)
    acc_c[...] = acc_c[...] + jnp.sum(oh, axis=1, keepdims=True)

    @pl.when(i == NPAD // PBLK - 1)
    def _fin():
        o_ref[...] = acc_s[...] / jnp.maximum(acc_c[...], 1.0)


def _pool(h, batch2d):
    nsteps = NPAD // PBLK
    return pl.pallas_call(
        _pool_body,
        grid=(nsteps,),
        in_specs=[
            pl.BlockSpec((PBLK, D), lambda i: (i, 0)),
            pl.BlockSpec((PBLK, 1), lambda i: (i, 0)),
        ],
        out_specs=pl.BlockSpec((G, D), lambda i: (0, 0)),
        out_shape=jax.ShapeDtypeStruct((G, D), jnp.float32),
        scratch_shapes=[
            pltpu.VMEM((G, D), jnp.float32),
            pltpu.VMEM((G, 128), jnp.float32),
        ],
        compiler_params=pltpu.CompilerParams(
            dimension_semantics=("arbitrary",)),
    )(h, batch2d)


def _head_body(hg_ref, wp_ref, bp_ref, o_ref):
    o_ref[0] = (jnp.dot(hg_ref[...], wp_ref[0],
                        preferred_element_type=jnp.float32)
                + bp_ref[0])


def _heads(hg, Wp_pad, bp_pad):
    return pl.pallas_call(
        _head_body,
        grid=(S, VPAD // VBLK),
        in_specs=[
            pl.BlockSpec((G, D), lambda s, v: (0, 0)),
            pl.BlockSpec((1, D, VBLK), lambda s, v: (s, 0, v)),
            pl.BlockSpec((1, 1, VBLK), lambda s, v: (s, 0, v)),
        ],
        out_specs=pl.BlockSpec((1, G, VBLK), lambda s, v: (s, 0, v)),
        out_shape=jax.ShapeDtypeStruct((S, G, VPAD), jnp.float32),
        compiler_params=pltpu.CompilerParams(
            dimension_semantics=("arbitrary", "arbitrary")),
    )(hg, Wp_pad, bp_pad)


def kernel(x, node_depth, edge_index, edge_attr, batch,
           emb_type, emb_attr, emb_depth,
           We, be, W1, b1, W2, b2, eps, Wp, bp):
    f32 = jnp.float32
    i32 = jnp.int32

    x0r = jnp.pad(x[:, 0].astype(i32), (0, NPAD - N)).reshape(NROWS, 128)
    x1r = jnp.pad(x[:, 1].astype(i32), (0, NPAD - N)).reshape(NROWS, 128)
    ddr = jnp.pad(node_depth[:, 0].astype(i32),
                  (0, NPAD - N)).reshape(NROWS, 128)
    src2d = jnp.pad(
        jnp.pad(edge_index[0].astype(i32), (0, EPAD - E)).reshape(ECHUNKS, CE),
        ((0, 8), (0, 0)))
    dst2d = jnp.pad(
        jnp.pad(edge_index[1].astype(i32), (0, EPAD - E),
                constant_values=N).reshape(ECHUNKS, CE),
        ((0, 8), (0, 0)), constant_values=N)
    ea_pad = jnp.pad(edge_attr.astype(f32), ((0, EPAD - E), (0, 0)))
    batch2d = jnp.pad(batch.astype(i32), (0, NPAD - N),
                      constant_values=G).reshape(NPAD, 1)

    h = _node_encoder(x0r, x1r, ddr, emb_type.astype(f32),
                      emb_attr.astype(f32), emb_depth.astype(f32))
    e_list = _edge_encoder(ea_pad, We.astype(f32), be.astype(f32))

    for l in range(NLAYER):
        aggp = _edge_aggregate(src2d, dst2d, e_list[l], h)
        epsv = (1.0 + eps[l]) * jnp.ones((1, D), f32)
        h = _mlp(h, aggp[0], aggp[1], epsv,
                 W1[l].astype(f32), b1[l][None, :].astype(f32),
                 W2[l].astype(f32), b2[l][None, :].astype(f32))

    hg = _pool(h, batch2d)
    Wp_pad = jnp.pad(Wp.astype(f32), ((0, 0), (0, 0), (0, VPAD - V)))
    bp_pad = jnp.pad(bp.astype(f32), ((0, 0), (0, VPAD - V))).reshape(S, 1, VPAD)
    preds = _heads(hg, Wp_pad, bp_pad)
    return preds[:, :, :V]


# speedup vs baseline: 2.3379x; 1.0072x over previous
import jax
import jax.numpy as jnp
from jax import lax
from jax.experimental import pallas as pl
from jax.experimental.pallas import tpu as pltpu
from jax.experimental.pallas import tpu_sc as plsc

N = 10000
E = 320000
D = 128
DE = 16
NLAYER = 4
G = 128
S = 5
V = 5002

NC = 2
NS = 16
NW = NC * NS

NPAD = 10240
NROWS = NPAD // 128
EPAD = 327680
ER = EPAD // 128
RPW = ER // NW
CE = 64
ECHUNKS = EPAD // CE
CPW = ECHUNKS // NW
CBLK = 16
IB = CBLK + 8
NSUPER = CPW // (2 * CBLK)
SROW = NPAD // NS

VPAD = 5120
VBLK = 640

_mesh = plsc.VectorSubcoreMesh(core_axis_name="c", subcore_axis_name="s")


def _enc_body(x0_hbm, x1_hbm, dd_hbm, ttype, tattr, tdepth, h0_hbm,
              ix0, ix1, ixd, b0, b1, b2, sem):
    c = lax.axis_index("c")
    s = lax.axis_index("s")
    w = s * NC + c
    start = jnp.where(w < 16, 3 * w, 48 + 2 * (w - 16))
    count = jnp.where(w < 16, 3, 2)

    def do_row(j, carry):
        row = start + j
        pltpu.sync_copy(x0_hbm.at[row], ix0.at[0])
        pltpu.sync_copy(x1_hbm.at[row], ix1.at[0])
        pltpu.sync_copy(dd_hbm.at[row], ixd.at[0])
        cp0 = pltpu.async_copy(ttype.at[ix0.at[0]], b0, sem)
        cp0.wait()
        cp1 = pltpu.async_copy(tattr.at[ix1.at[0]], b1, sem)
        cp1.wait()
        cp2 = pltpu.async_copy(tdepth.at[ixd.at[0]], b2, sem)
        cp2.wait()

        def srow(r, carry2):
            for jj in range(8):
                dsl = pl.ds(jj * 16, 16)
                b0[r, dsl] = b0[r, dsl] + b1[r, dsl] + b2[r, dsl]
            return carry2

        lax.fori_loop(0, 128, srow, 0)
        pltpu.sync_copy(b0, h0_hbm.at[pl.ds(row * 128, 128)])
        return carry

    lax.fori_loop(0, count, do_row, 0)


def _node_encoder(x0r, x1r, ddr, emb_type, emb_attr, emb_depth):
    f = pl.kernel(
        _enc_body,
        out_type=jax.ShapeDtypeStruct((NPAD, D), jnp.float32),
        mesh=_mesh,
        scratch_types=[
            pltpu.VMEM((1, 128), jnp.int32),
            pltpu.VMEM((1, 128), jnp.int32),
            pltpu.VMEM((1, 128), jnp.int32),
            pltpu.VMEM((128, D), jnp.float32),
            pltpu.VMEM((128, D), jnp.float32),
            pltpu.VMEM((128, D), jnp.float32),
            pltpu.SemaphoreType.DMA,
        ],
    )
    return f(x0r, x1r, ddr, emb_type, emb_attr, emb_depth)


def _edge_body(src_hbm, dst_hbm, e_hbm, h_hbm, aggp_hbm,
               s0b, s1b, d0b, d1b, eb, hA, hB, ob, acc,
               sem_e, sem_g0, sem_g1, sem_s):
    c = lax.axis_index("c")
    s = lax.axis_index("s")
    w = s * NC + c
    w0 = w * CPW
    base = s * SROW

    sbufs = (s0b, s1b)
    dbufs = (d0b, d1b)

    def wait_bytes(dst, sem):
        pltpu.make_async_copy(e_hbm.at[pl.ds(0, CE)], dst, sem).wait()

    def compute(obuf, hbuf, ebuf):
        def erow(r, carry2):
            for jj in range(8):
                dsl = pl.ds(jj * 16, 16)
                obuf[r, dsl] = jnp.maximum(ebuf[r, dsl] + hbuf[r, dsl], 0.0)
            return carry2

        lax.fori_loop(0, CE, erow, 0)

    def zrow(r, carry):
        for jj in range(8):
            ob[r, pl.ds(jj * 16, 16)] = jnp.zeros((16,), jnp.float32)
        return carry

    lax.fori_loop(0, CE, zrow, 0)
    for t in range(SROW // CE):
        pltpu.async_copy(ob, acc.at[pl.ds(base + t * CE, CE)], sem_s)
    pltpu.sync_copy(src_hbm.at[pl.ds(w0, IB)], s0b)
    pltpu.sync_copy(dst_hbm.at[pl.ds(w0, IB)], d0b)
    pltpu.async_copy(h_hbm.at[s0b.at[0]], hA, sem_g0)
    pltpu.async_copy(h_hbm.at[s0b.at[1]], hB, sem_g1)
    pltpu.async_copy(e_hbm.at[pl.ds(w0 * CE, CE)], eb, sem_e)
    pltpu.make_async_copy(e_hbm.at[pl.ds(0, SROW)],
                          acc.at[pl.ds(base, SROW)], sem_s).wait()
    plsc.subcore_barrier()

    def super_body(sp, carry):
        for par in range(2):
            sbuf = sbufs[par]
            dbuf = dbufs[par]
            bk = sp * 2 + par
            lb = bk * CBLK

            def load_idx():
                pltpu.sync_copy(src_hbm.at[pl.ds(w0 + lb, IB)], sbuf)
                pltpu.sync_copy(dst_hbm.at[pl.ds(w0 + lb, IB)], dbuf)

            if par == 0:
                pl.when(sp > 0)(load_idx)
            else:
                load_idx()

            def body(i2, carry2):
                for b, (hX, sg) in enumerate(((hA, sem_g0), (hB, sem_g1))):
                    jl = 2 * i2 + b
                    lc = lb + jl
                    gc = w0 + lc

                    wait_bytes(hX, sg)
                    wait_bytes(eb, sem_e)
                    pl.when(lc >= 1)(lambda: wait_bytes(ob, sem_s))
                    compute(ob, hX, eb)
                    pltpu.async_copy(ob, acc.at[dbuf.at[jl]], sem_s, add=True)

                    def next_e(gc=gc):
                        pltpu.async_copy(
                            e_hbm.at[pl.ds((gc + 1) * CE, CE)], eb, sem_e)

                    pl.when(lc < CPW - 1)(next_e)

                    def next_g(jl=jl, hX=hX, sg=sg, sbuf=sbuf):
                        pltpu.async_copy(h_hbm.at[sbuf.at[jl + 2]], hX, sg)

                    pl.when(lc < CPW - 2)(next_g)
                return carry2

            lax.fori_loop(0, CBLK // 2, body, 0)
        return carry

    lax.fori_loop(0, NSUPER, super_body, 0)
    wait_bytes(ob, sem_s)
    plsc.subcore_barrier()
    pltpu.sync_copy(acc.at[pl.ds(base, SROW)],
                    aggp_hbm.at[c, pl.ds(base, SROW)])


def _edge_aggregate(src2d, dst2d, e_l, h):
    f = pl.kernel(
        _edge_body,
        out_type=jax.ShapeDtypeStruct((NC, NPAD, D), jnp.float32),
        mesh=_mesh,
        scratch_types=[
            pltpu.VMEM((IB, CE), jnp.int32),
            pltpu.VMEM((IB, CE), jnp.int32),
            pltpu.VMEM((IB, CE), jnp.int32),
            pltpu.VMEM((IB, CE), jnp.int32),
            pltpu.VMEM((CE, D), jnp.float32),
            pltpu.VMEM((CE, D), jnp.float32),
            pltpu.VMEM((CE, D), jnp.float32),
            pltpu.VMEM((CE, D), jnp.float32),
            pltpu.VMEM_SHARED((NPAD, D), jnp.float32),
            pltpu.SemaphoreType.DMA,
            pltpu.SemaphoreType.DMA,
            pltpu.SemaphoreType.DMA,
            pltpu.SemaphoreType.DMA,
        ],
    )
    return f(src2d, dst2d, e_l, h)


EBLK = 2048


def _ee_body(ea_ref, we_ref, be_ref, o0, o1, o2, o3):
    a = ea_ref[...]
    outs = (o0, o1, o2, o3)
    for l in range(NLAYER):
        outs[l][...] = (jnp.dot(a, we_ref[l],
                                preferred_element_type=jnp.float32)
                        + be_ref[l:l + 1, :])


def _edge_encoder(ea_pad, We, be):
    nsteps = EPAD // EBLK
    return pl.pallas_call(
        _ee_body,
        grid=(nsteps,),
        in_specs=[
            pl.BlockSpec((EBLK, DE), lambda i: (i, 0)),
            pl.BlockSpec((NLAYER, DE, D), lambda i: (0, 0, 0)),
            pl.BlockSpec((NLAYER, D), lambda i: (0, 0)),
        ],
        out_specs=tuple(pl.BlockSpec((EBLK, D), lambda i: (i, 0))
                        for _ in range(NLAYER)),
        out_shape=tuple(jax.ShapeDtypeStruct((EPAD, D), jnp.float32)
                        for _ in range(NLAYER)),
        compiler_params=pltpu.CompilerParams(
            dimension_semantics=("arbitrary",)),
    )(ea_pad, We, be)


MBLK = 1024


def _mlp_body(h_ref, a0_ref, a1_ref, ep_ref, w1_ref, b1_ref, w2_ref, b2_ref,
              o_ref):
    t = h_ref[...] * ep_ref[...] + a0_ref[...] + a1_ref[...]
    z = jnp.maximum(
        jnp.dot(t, w1_ref[...], preferred_element_type=jnp.float32)
        + b1_ref[...], 0.0)
    o_ref[...] = jnp.maximum(
        jnp.dot(z, w2_ref[...], preferred_element_type=jnp.float32)
        + b2_ref[...], 0.0)


def _mlp(h, a0, a1, epsv, W1l, b1l, W2l, b2l):
    nsteps = NPAD // MBLK
    return pl.pallas_call(
        _mlp_body,
        grid=(nsteps,),
        in_specs=[
            pl.BlockSpec((MBLK, D), lambda i: (i, 0)),
            pl.BlockSpec((MBLK, D), lambda i: (i, 0)),
            pl.BlockSpec((MBLK, D), lambda i: (i, 0)),
            pl.BlockSpec((1, D), lambda i: (0, 0)),
            pl.BlockSpec((D, D), lambda i: (0, 0)),
            pl.BlockSpec((1, D), lambda i: (0, 0)),
            pl.BlockSpec((D, D), lambda i: (0, 0)),
            pl.BlockSpec((1, D), lambda i: (0, 0)),
        ],
        out_specs=pl.BlockSpec((MBLK, D), lambda i: (i, 0)),
        out_shape=jax.ShapeDtypeStruct((NPAD, D), jnp.float32),
        compiler_params=pltpu.CompilerParams(
            dimension_semantics=("arbitrary",)),
    )(h, a0, a1, epsv, W1l, b1l, W2l, b2l)


PBLK = 1024


def _pool_body(h_ref, b_ref, o_ref, acc_s, acc_c):
    i = pl.program_id(0)

    @pl.when(i == 0)
    def _init():
        acc_s[...] = jnp.zeros_like(acc_s)
        acc_c[...] = jnp.zeros_like(acc_c)

    bt = b_ref[...].reshape(1, PBLK)
    gi = lax.broadcasted_iota(jnp.int32, (G, PBLK), 0)
    oh = (gi == bt).astype(jnp.float32)
    acc_s[...] += jnp.dot(oh, h_ref[...], preferred_element_type=jnp.float32)
    acc_c[...] = acc_c[...] + jnp.sum(oh, axis=1, keepdims=True)

    @pl.when(i == NPAD // PBLK - 1)
    def _fin():
        o_ref[...] = acc_s[...] / jnp.maximum(acc_c[...], 1.0)


def _pool(h, batch2d):
    nsteps = NPAD // PBLK
    return pl.pallas_call(
        _pool_body,
        grid=(nsteps,),
        in_specs=[
            pl.BlockSpec((PBLK, D), lambda i: (i, 0)),
            pl.BlockSpec((PBLK, 1), lambda i: (i, 0)),
        ],
        out_specs=pl.BlockSpec((G, D), lambda i: (0, 0)),
        out_shape=jax.ShapeDtypeStruct((G, D), jnp.float32),
        scratch_shapes=[
            pltpu.VMEM((G, D), jnp.float32),
            pltpu.VMEM((G, 128), jnp.float32),
        ],
        compiler_params=pltpu.CompilerParams(
            dimension_semantics=("arbitrary",)),
    )(h, batch2d)


def _head_body(hg_ref, wp_ref, bp_ref, o_ref):
    o_ref[0] = (jnp.dot(hg_ref[...], wp_ref[0],
                        preferred_element_type=jnp.float32)
                + bp_ref[0])


def _heads(hg, Wp_pad, bp_pad):
    return pl.pallas_call(
        _head_body,
        grid=(S, VPAD // VBLK),
        in_specs=[
            pl.BlockSpec((G, D), lambda s, v: (0, 0)),
            pl.BlockSpec((1, D, VBLK), lambda s, v: (s, 0, v)),
            pl.BlockSpec((1, 1, VBLK), lambda s, v: (s, 0, v)),
        ],
        out_specs=pl.BlockSpec((1, G, VBLK), lambda s, v: (s, 0, v)),
        out_shape=jax.ShapeDtypeStruct((S, G, VPAD), jnp.float32),
        compiler_params=pltpu.CompilerParams(
            dimension_semantics=("arbitrary", "arbitrary")),
    )(hg, Wp_pad, bp_pad)


def kernel(x, node_depth, edge_index, edge_attr, batch,
           emb_type, emb_attr, emb_depth,
           We, be, W1, b1, W2, b2, eps, Wp, bp):
    f32 = jnp.float32
    i32 = jnp.int32

    x0r = jnp.pad(x[:, 0].astype(i32), (0, NPAD - N)).reshape(NROWS, 128)
    x1r = jnp.pad(x[:, 1].astype(i32), (0, NPAD - N)).reshape(NROWS, 128)
    ddr = jnp.pad(node_depth[:, 0].astype(i32),
                  (0, NPAD - N)).reshape(NROWS, 128)
    src2d = jnp.pad(
        jnp.pad(edge_index[0].astype(i32), (0, EPAD - E)).reshape(ECHUNKS, CE),
        ((0, 8), (0, 0)))
    dst2d = jnp.pad(
        jnp.pad(edge_index[1].astype(i32), (0, EPAD - E),
                constant_values=N).reshape(ECHUNKS, CE),
        ((0, 8), (0, 0)), constant_values=N)
    ea_pad = jnp.pad(edge_attr.astype(f32), ((0, EPAD - E), (0, 0)))
    batch2d = jnp.pad(batch.astype(i32), (0, NPAD - N),
                      constant_values=G).reshape(NPAD, 1)

    h = _node_encoder(x0r, x1r, ddr, emb_type.astype(f32),
                      emb_attr.astype(f32), emb_depth.astype(f32))
    e_list = _edge_encoder(ea_pad, We.astype(f32), be.astype(f32))

    for l in range(NLAYER):
        aggp = _edge_aggregate(src2d, dst2d, e_list[l], h)
        epsv = (1.0 + eps[l]) * jnp.ones((1, D), f32)
        h = _mlp(h, aggp[0], aggp[1], epsv,
                 W1[l].astype(f32), b1[l][None, :].astype(f32),
                 W2[l].astype(f32), b2[l][None, :].astype(f32))

    hg = _pool(h, batch2d)
    Wp_pad = jnp.pad(Wp.astype(f32), ((0, 0), (0, 0), (0, VPAD - V)))
    bp_pad = jnp.pad(bp.astype(f32), ((0, 0), (0, VPAD - V))).reshape(S, 1, VPAD)
    preds = _heads(hg, Wp_pad, bp_pad)
    return preds[:, :, :V]


# speedup vs baseline: 2.3757x; 1.0161x over previous
import jax
import jax.numpy as jnp
from jax import lax
from jax.experimental import pallas as pl
from jax.experimental.pallas import tpu as pltpu
from jax.experimental.pallas import tpu_sc as plsc

N = 10000
E = 320000
D = 128
DE = 16
NLAYER = 4
G = 128
S = 5
V = 5002

NC = 2
NS = 16
NW = NC * NS

NPAD = 10240
NROWS = NPAD // 128
EPAD = 327680
ER = EPAD // 128
RPW = ER // NW
CE = 64
ECHUNKS = EPAD // CE
CPW = ECHUNKS // NW
CBLK = 16
IB = CBLK + 8
NSUPER = CPW // (2 * CBLK)
SROW = NPAD // NS

VPAD = 5120
VBLK = 640

_mesh = plsc.VectorSubcoreMesh(core_axis_name="c", subcore_axis_name="s")


def _enc_body(x0_hbm, x1_hbm, dd_hbm, ttype, tattr, tdepth, h0_hbm,
              ix0, ix1, ixd, b0, b1, b2, sem):
    c = lax.axis_index("c")
    s = lax.axis_index("s")
    w = s * NC + c
    start = jnp.where(w < 16, 3 * w, 48 + 2 * (w - 16))
    count = jnp.where(w < 16, 3, 2)

    def do_row(j, carry):
        row = start + j
        pltpu.sync_copy(x0_hbm.at[row], ix0.at[0])
        pltpu.sync_copy(x1_hbm.at[row], ix1.at[0])
        pltpu.sync_copy(dd_hbm.at[row], ixd.at[0])
        cp0 = pltpu.async_copy(ttype.at[ix0.at[0]], b0, sem)
        cp0.wait()
        cp1 = pltpu.async_copy(tattr.at[ix1.at[0]], b1, sem)
        cp1.wait()
        cp2 = pltpu.async_copy(tdepth.at[ixd.at[0]], b2, sem)
        cp2.wait()

        def srow(r, carry2):
            for jj in range(8):
                dsl = pl.ds(jj * 16, 16)
                b0[r, dsl] = b0[r, dsl] + b1[r, dsl] + b2[r, dsl]
            return carry2

        lax.fori_loop(0, 128, srow, 0)
        pltpu.sync_copy(b0, h0_hbm.at[pl.ds(row * 128, 128)])
        return carry

    lax.fori_loop(0, count, do_row, 0)


def _node_encoder(x0r, x1r, ddr, emb_type, emb_attr, emb_depth):
    f = pl.kernel(
        _enc_body,
        out_type=jax.ShapeDtypeStruct((NPAD, D), jnp.float32),
        mesh=_mesh,
        scratch_types=[
            pltpu.VMEM((1, 128), jnp.int32),
            pltpu.VMEM((1, 128), jnp.int32),
            pltpu.VMEM((1, 128), jnp.int32),
            pltpu.VMEM((128, D), jnp.float32),
            pltpu.VMEM((128, D), jnp.float32),
            pltpu.VMEM((128, D), jnp.float32),
            pltpu.SemaphoreType.DMA,
        ],
    )
    return f(x0r, x1r, ddr, emb_type, emb_attr, emb_depth)


def _edge_body(src_hbm, dst_hbm, e_hbm, h_hbm, aggp_hbm,
               s0b, s1b, d0b, d1b, eb, hA, hB, ob, acc,
               sem_e, sem_g0, sem_g1, sem_s):
    c = lax.axis_index("c")
    s = lax.axis_index("s")
    w = s * NC + c
    w0 = w * CPW
    base = s * SROW

    sbufs = (s0b, s1b)
    dbufs = (d0b, d1b)

    def wait_bytes(dst, sem):
        pltpu.make_async_copy(e_hbm.at[pl.ds(0, CE)], dst, sem).wait()

    def compute(obuf, hbuf, ebuf):
        def erow(r, carry2):
            for jj in range(8):
                dsl = pl.ds(jj * 16, 16)
                obuf[r, dsl] = jnp.maximum(ebuf[r, dsl] + hbuf[r, dsl], 0.0)
            return carry2

        lax.fori_loop(0, CE, erow, 0)

    def zrow(r, carry):
        for jj in range(8):
            ob[r, pl.ds(jj * 16, 16)] = jnp.zeros((16,), jnp.float32)
        return carry

    lax.fori_loop(0, CE, zrow, 0)
    for t in range(SROW // CE):
        pltpu.async_copy(ob, acc.at[pl.ds(base + t * CE, CE)], sem_s)
    pltpu.sync_copy(src_hbm.at[pl.ds(w0, IB)], s0b)
    pltpu.sync_copy(dst_hbm.at[pl.ds(w0, IB)], d0b)
    pltpu.async_copy(h_hbm.at[s0b.at[0]], hA, sem_g0)
    pltpu.async_copy(h_hbm.at[s0b.at[1]], hB, sem_g1)
    pltpu.async_copy(e_hbm.at[pl.ds(w0 * CE, CE)], eb, sem_e)
    pltpu.make_async_copy(e_hbm.at[pl.ds(0, SROW)],
                          acc.at[pl.ds(base, SROW)], sem_s).wait()
    plsc.subcore_barrier()

    def super_body(sp, carry):
        for par in range(2):
            sbuf = sbufs[par]
            dbuf = dbufs[par]
            bk = sp * 2 + par
            lb = bk * CBLK

            def load_idx():
                pltpu.sync_copy(src_hbm.at[pl.ds(w0 + lb, IB)], sbuf)
                pltpu.sync_copy(dst_hbm.at[pl.ds(w0 + lb, IB)], dbuf)

            if par == 0:
                pl.when(sp > 0)(load_idx)
            else:
                load_idx()

            def body(i2, carry2):
                for b, (hX, sg) in enumerate(((hA, sem_g0), (hB, sem_g1))):
                    jl = 2 * i2 + b
                    lc = lb + jl
                    gc = w0 + lc

                    wait_bytes(hX, sg)
                    wait_bytes(eb, sem_e)
                    pl.when(lc >= 1)(lambda: wait_bytes(ob, sem_s))
                    compute(ob, hX, eb)
                    pltpu.async_copy(ob, acc.at[dbuf.at[jl]], sem_s, add=True)

                    def next_e(gc=gc):
                        pltpu.async_copy(
                            e_hbm.at[pl.ds((gc + 1) * CE, CE)], eb, sem_e)

                    pl.when(lc < CPW - 1)(next_e)

                    def next_g(jl=jl, hX=hX, sg=sg, sbuf=sbuf):
                        pltpu.async_copy(h_hbm.at[sbuf.at[jl + 2]], hX, sg)

                    pl.when(lc < CPW - 2)(next_g)
                return carry2

            lax.fori_loop(0, CBLK // 2, body, 0)
        return carry

    lax.fori_loop(0, NSUPER, super_body, 0)
    wait_bytes(ob, sem_s)
    plsc.subcore_barrier()
    pltpu.sync_copy(acc.at[pl.ds(base, SROW)],
                    aggp_hbm.at[c, pl.ds(base, SROW)])


def _edge_aggregate(src2d, dst2d, e_l, h):
    f = pl.kernel(
        _edge_body,
        out_type=jax.ShapeDtypeStruct((NC, NPAD, D), jnp.float32),
        mesh=_mesh,
        scratch_types=[
            pltpu.VMEM((IB, CE), jnp.int32),
            pltpu.VMEM((IB, CE), jnp.int32),
            pltpu.VMEM((IB, CE), jnp.int32),
            pltpu.VMEM((IB, CE), jnp.int32),
            pltpu.VMEM((CE, D), jnp.float32),
            pltpu.VMEM((CE, D), jnp.float32),
            pltpu.VMEM((CE, D), jnp.float32),
            pltpu.VMEM((CE, D), jnp.float32),
            pltpu.VMEM_SHARED((NPAD, D), jnp.float32),
            pltpu.SemaphoreType.DMA,
            pltpu.SemaphoreType.DMA,
            pltpu.SemaphoreType.DMA,
            pltpu.SemaphoreType.DMA,
        ],
    )
    return f(src2d, dst2d, e_l, h)


EBLK = 2048


def _ee_body(ea_ref, we_ref, be_ref, o_ref):
    o_ref[...] = (jnp.dot(ea_ref[...], we_ref[...],
                          preferred_element_type=jnp.float32)
                  + be_ref[...])


def _edge_encoder(ea_pad, We_l, be_l):
    nsteps = EPAD // EBLK
    return pl.pallas_call(
        _ee_body,
        grid=(nsteps,),
        in_specs=[
            pl.BlockSpec((EBLK, DE), lambda i: (i, 0)),
            pl.BlockSpec((DE, D), lambda i: (0, 0)),
            pl.BlockSpec((1, D), lambda i: (0, 0)),
        ],
        out_specs=pl.BlockSpec((EBLK, D), lambda i: (i, 0)),
        out_shape=jax.ShapeDtypeStruct((EPAD, D), jnp.float32),
        compiler_params=pltpu.CompilerParams(
            dimension_semantics=("arbitrary",)),
    )(ea_pad, We_l, be_l)


MBLK = 1024


def _mlp_body(h_ref, a0_ref, a1_ref, ep_ref, w1_ref, b1_ref, w2_ref, b2_ref,
              o_ref):
    t = h_ref[...] * ep_ref[...] + a0_ref[...] + a1_ref[...]
    z = jnp.maximum(
        jnp.dot(t, w1_ref[...], preferred_element_type=jnp.float32)
        + b1_ref[...], 0.0)
    o_ref[...] = jnp.maximum(
        jnp.dot(z, w2_ref[...], preferred_element_type=jnp.float32)
        + b2_ref[...], 0.0)


def _mlp(h, a0, a1, epsv, W1l, b1l, W2l, b2l):
    nsteps = NPAD // MBLK
    return pl.pallas_call(
        _mlp_body,
        grid=(nsteps,),
        in_specs=[
            pl.BlockSpec((MBLK, D), lambda i: (i, 0)),
            pl.BlockSpec((MBLK, D), lambda i: (i, 0)),
            pl.BlockSpec((MBLK, D), lambda i: (i, 0)),
            pl.BlockSpec((1, D), lambda i: (0, 0)),
            pl.BlockSpec((D, D), lambda i: (0, 0)),
            pl.BlockSpec((1, D), lambda i: (0, 0)),
            pl.BlockSpec((D, D), lambda i: (0, 0)),
            pl.BlockSpec((1, D), lambda i: (0, 0)),
        ],
        out_specs=pl.BlockSpec((MBLK, D), lambda i: (i, 0)),
        out_shape=jax.ShapeDtypeStruct((NPAD, D), jnp.float32),
        compiler_params=pltpu.CompilerParams(
            dimension_semantics=("arbitrary",)),
    )(h, a0, a1, epsv, W1l, b1l, W2l, b2l)


PBLK = 1024


def _pool_body(h_ref, b_ref, o_ref, acc_s, acc_c):
    i = pl.program_id(0)

    @pl.when(i == 0)
    def _init():
        acc_s[...] = jnp.zeros_like(acc_s)
        acc_c[...] = jnp.zeros_like(acc_c)

    bt = b_ref[...].reshape(1, PBLK)
    gi = lax.broadcasted_iota(jnp.int32, (G, PBLK), 0)
    oh = (gi == bt).astype(jnp.float32)
    acc_s[...] += jnp.dot(oh, h_ref[...], preferred_element_type=jnp.float32)
    acc_c[...] = acc_c[...] + jnp.sum(oh, axis=1, keepdims=True)

    @pl.when(i == NPAD // PBLK - 1)
    def _fin():
        o_ref[...] = acc_s[...] / jnp.maximum(acc_c[...], 1.0)


def _pool(h, batch2d):
    nsteps = NPAD // PBLK
    return pl.pallas_call(
        _pool_body,
        grid=(nsteps,),
        in_specs=[
            pl.BlockSpec((PBLK, D), lambda i: (i, 0)),
            pl.BlockSpec((PBLK, 1), lambda i: (i, 0)),
        ],
        out_specs=pl.BlockSpec((G, D), lambda i: (0, 0)),
        out_shape=jax.ShapeDtypeStruct((G, D), jnp.float32),
        scratch_shapes=[
            pltpu.VMEM((G, D), jnp.float32),
            pltpu.VMEM((G, 128), jnp.float32),
        ],
        compiler_params=pltpu.CompilerParams(
            dimension_semantics=("arbitrary",)),
    )(h, batch2d)


def _head_body(hg_ref, wp_ref, bp_ref, o_ref):
    o_ref[0] = (jnp.dot(hg_ref[...], wp_ref[0],
                        preferred_element_type=jnp.float32)
                + bp_ref[0])


def _heads(hg, Wp_pad, bp_pad):
    return pl.pallas_call(
        _head_body,
        grid=(S, VPAD // VBLK),
        in_specs=[
            pl.BlockSpec((G, D), lambda s, v: (0, 0)),
            pl.BlockSpec((1, D, VBLK), lambda s, v: (s, 0, v)),
            pl.BlockSpec((1, 1, VBLK), lambda s, v: (s, 0, v)),
        ],
        out_specs=pl.BlockSpec((1, G, VBLK), lambda s, v: (s, 0, v)),
        out_shape=jax.ShapeDtypeStruct((S, G, VPAD), jnp.float32),
        compiler_params=pltpu.CompilerParams(
            dimension_semantics=("arbitrary", "arbitrary")),
    )(hg, Wp_pad, bp_pad)


def kernel(x, node_depth, edge_index, edge_attr, batch,
           emb_type, emb_attr, emb_depth,
           We, be, W1, b1, W2, b2, eps, Wp, bp):
    f32 = jnp.float32
    i32 = jnp.int32

    x0r = jnp.pad(x[:, 0].astype(i32), (0, NPAD - N)).reshape(NROWS, 128)
    x1r = jnp.pad(x[:, 1].astype(i32), (0, NPAD - N)).reshape(NROWS, 128)
    ddr = jnp.pad(node_depth[:, 0].astype(i32),
                  (0, NPAD - N)).reshape(NROWS, 128)
    src2d = jnp.pad(
        jnp.pad(edge_index[0].astype(i32), (0, EPAD - E)).reshape(ECHUNKS, CE),
        ((0, 8), (0, 0)))
    dst2d = jnp.pad(
        jnp.pad(edge_index[1].astype(i32), (0, EPAD - E),
                constant_values=N).reshape(ECHUNKS, CE),
        ((0, 8), (0, 0)), constant_values=N)
    ea_pad = jnp.pad(edge_attr.astype(f32), ((0, EPAD - E), (0, 0)))
    batch2d = jnp.pad(batch.astype(i32), (0, NPAD - N),
                      constant_values=G).reshape(NPAD, 1)

    h = _node_encoder(x0r, x1r, ddr, emb_type.astype(f32),
                      emb_attr.astype(f32), emb_depth.astype(f32))

    for l in range(NLAYER):
        e_l = _edge_encoder(ea_pad, We[l].astype(f32),
                            be[l][None, :].astype(f32))
        aggp = _edge_aggregate(src2d, dst2d, e_l, h)
        epsv = (1.0 + eps[l]) * jnp.ones((1, D), f32)
        h = _mlp(h, aggp[0], aggp[1], epsv,
                 W1[l].astype(f32), b1[l][None, :].astype(f32),
                 W2[l].astype(f32), b2[l][None, :].astype(f32))

    hg = _pool(h, batch2d)
    Wp_pad = jnp.pad(Wp.astype(f32), ((0, 0), (0, 0), (0, VPAD - V)))
    bp_pad = jnp.pad(bp.astype(f32), ((0, 0), (0, VPAD - V))).reshape(S, 1, VPAD)
    preds = _heads(hg, Wp_pad, bp_pad)
    return preds[:, :, :V]


# speedup vs baseline: 2.3762x; 1.0002x over previous
import jax
import jax.numpy as jnp
from jax import lax
from jax.experimental import pallas as pl
from jax.experimental.pallas import tpu as pltpu
from jax.experimental.pallas import tpu_sc as plsc

N = 10000
E = 320000
D = 128
DE = 16
NLAYER = 4
G = 128
S = 5
V = 5002

NC = 2
NS = 16
NW = NC * NS

NPAD = 10240
NROWS = NPAD // 128
EPAD = 327680
ER = EPAD // 128
RPW = ER // NW
CE = 64
ECHUNKS = EPAD // CE
CPW = ECHUNKS // NW
CBLK = 16
IB = CBLK + 8
NSUPER = CPW // (2 * CBLK)
SROW = NPAD // NS

VPAD = 5120
VBLK = 640

_mesh = plsc.VectorSubcoreMesh(core_axis_name="c", subcore_axis_name="s")


def _enc_body(x0_hbm, x1_hbm, dd_hbm, ttype, tattr, tdepth, h0_hbm,
              ix0, ix1, ixd, b0, b1, b2, sem):
    c = lax.axis_index("c")
    s = lax.axis_index("s")
    w = s * NC + c
    start = jnp.where(w < 16, 3 * w, 48 + 2 * (w - 16))
    count = jnp.where(w < 16, 3, 2)

    def do_row(j, carry):
        row = start + j
        pltpu.sync_copy(x0_hbm.at[row], ix0.at[0])
        pltpu.sync_copy(x1_hbm.at[row], ix1.at[0])
        pltpu.sync_copy(dd_hbm.at[row], ixd.at[0])
        cp0 = pltpu.async_copy(ttype.at[ix0.at[0]], b0, sem)
        cp0.wait()
        cp1 = pltpu.async_copy(tattr.at[ix1.at[0]], b1, sem)
        cp1.wait()
        cp2 = pltpu.async_copy(tdepth.at[ixd.at[0]], b2, sem)
        cp2.wait()

        def srow(r, carry2):
            for jj in range(8):
                dsl = pl.ds(jj * 16, 16)
                b0[r, dsl] = b0[r, dsl] + b1[r, dsl] + b2[r, dsl]
            return carry2

        lax.fori_loop(0, 128, srow, 0)
        pltpu.sync_copy(b0, h0_hbm.at[pl.ds(row * 128, 128)])
        return carry

    lax.fori_loop(0, count, do_row, 0)


def _node_encoder(x0r, x1r, ddr, emb_type, emb_attr, emb_depth):
    f = pl.kernel(
        _enc_body,
        out_type=jax.ShapeDtypeStruct((NPAD, D), jnp.float32),
        mesh=_mesh,
        scratch_types=[
            pltpu.VMEM((1, 128), jnp.int32),
            pltpu.VMEM((1, 128), jnp.int32),
            pltpu.VMEM((1, 128), jnp.int32),
            pltpu.VMEM((128, D), jnp.float32),
            pltpu.VMEM((128, D), jnp.float32),
            pltpu.VMEM((128, D), jnp.float32),
            pltpu.SemaphoreType.DMA,
        ],
    )
    return f(x0r, x1r, ddr, emb_type, emb_attr, emb_depth)


def _edge_body(src_hbm, dst_hbm, e_hbm, h_hbm, aggp_hbm,
               s0b, s1b, d0b, d1b, eb, hA, hB, ob, acc,
               sem_e, sem_g0, sem_g1, sem_s):
    c = lax.axis_index("c")
    s = lax.axis_index("s")
    w = s * NC + c
    w0 = w * CPW
    base = s * SROW

    sbufs = (s0b, s1b)
    dbufs = (d0b, d1b)

    def wait_bytes(dst, sem):
        pltpu.make_async_copy(e_hbm.at[pl.ds(0, CE)], dst, sem).wait()

    def compute(obuf, hbuf, ebuf):
        def erow(r4, carry2):
            r0 = r4 * 4
            for rr in range(4):
                r = r0 + rr
                for jj in range(8):
                    dsl = pl.ds(jj * 16, 16)
                    obuf[r, dsl] = jnp.maximum(
                        ebuf[r, dsl] + hbuf[r, dsl], 0.0)
            return carry2

        lax.fori_loop(0, CE // 4, erow, 0)

    def zrow(r, carry):
        for jj in range(8):
            ob[r, pl.ds(jj * 16, 16)] = jnp.zeros((16,), jnp.float32)
        return carry

    lax.fori_loop(0, CE, zrow, 0)
    for t in range(SROW // CE):
        pltpu.async_copy(ob, acc.at[pl.ds(base + t * CE, CE)], sem_s)
    pltpu.sync_copy(src_hbm.at[pl.ds(w0, IB)], s0b)
    pltpu.sync_copy(dst_hbm.at[pl.ds(w0, IB)], d0b)
    pltpu.async_copy(h_hbm.at[s0b.at[0]], hA, sem_g0)
    pltpu.async_copy(h_hbm.at[s0b.at[1]], hB, sem_g1)
    pltpu.async_copy(e_hbm.at[pl.ds(w0 * CE, CE)], eb, sem_e)
    pltpu.make_async_copy(e_hbm.at[pl.ds(0, SROW)],
                          acc.at[pl.ds(base, SROW)], sem_s).wait()
    plsc.subcore_barrier()

    def super_body(sp, carry):
        for par in range(2):
            sbuf = sbufs[par]
            dbuf = dbufs[par]
            bk = sp * 2 + par
            lb = bk * CBLK

            def load_idx():
                pltpu.sync_copy(src_hbm.at[pl.ds(w0 + lb, IB)], sbuf)
                pltpu.sync_copy(dst_hbm.at[pl.ds(w0 + lb, IB)], dbuf)

            if par == 0:
                pl.when(sp > 0)(load_idx)
            else:
                load_idx()

            def body(i2, carry2):
                for b, (hX, sg) in enumerate(((hA, sem_g0), (hB, sem_g1))):
                    jl = 2 * i2 + b
                    lc = lb + jl
                    gc = w0 + lc

                    wait_bytes(hX, sg)
                    wait_bytes(eb, sem_e)
                    pl.when(lc >= 1)(lambda: wait_bytes(ob, sem_s))
                    compute(ob, hX, eb)
                    pltpu.async_copy(ob, acc.at[dbuf.at[jl]], sem_s, add=True)

                    def next_e(gc=gc):
                        pltpu.async_copy(
                            e_hbm.at[pl.ds((gc + 1) * CE, CE)], eb, sem_e)

                    pl.when(lc < CPW - 1)(next_e)

                    def next_g(jl=jl, hX=hX, sg=sg, sbuf=sbuf):
                        pltpu.async_copy(h_hbm.at[sbuf.at[jl + 2]], hX, sg)

                    pl.when(lc < CPW - 2)(next_g)
                return carry2

            lax.fori_loop(0, CBLK // 2, body, 0)
        return carry

    lax.fori_loop(0, NSUPER, super_body, 0)
    wait_bytes(ob, sem_s)
    plsc.subcore_barrier()
    pltpu.sync_copy(acc.at[pl.ds(base, SROW)],
                    aggp_hbm.at[c, pl.ds(base, SROW)])


def _edge_aggregate(src2d, dst2d, e_l, h):
    f = pl.kernel(
        _edge_body,
        out_type=jax.ShapeDtypeStruct((NC, NPAD, D), jnp.float32),
        mesh=_mesh,
        scratch_types=[
            pltpu.VMEM((IB, CE), jnp.int32),
            pltpu.VMEM((IB, CE), jnp.int32),
            pltpu.VMEM((IB, CE), jnp.int32),
            pltpu.VMEM((IB, CE), jnp.int32),
            pltpu.VMEM((CE, D), jnp.float32),
            pltpu.VMEM((CE, D), jnp.float32),
            pltpu.VMEM((CE, D), jnp.float32),
            pltpu.VMEM((CE, D), jnp.float32),
            pltpu.VMEM_SHARED((NPAD, D), jnp.float32),
            pltpu.SemaphoreType.DMA,
            pltpu.SemaphoreType.DMA,
            pltpu.SemaphoreType.DMA,
            pltpu.SemaphoreType.DMA,
        ],
    )
    return f(src2d, dst2d, e_l, h)


EBLK = 2048


def _ee_body(ea_ref, we_ref, be_ref, o_ref):
    o_ref[...] = (jnp.dot(ea_ref[...], we_ref[...],
                          preferred_element_type=jnp.float32)
                  + be_ref[...])


def _edge_encoder(ea_pad, We_l, be_l):
    nsteps = EPAD // EBLK
    return pl.pallas_call(
        _ee_body,
        grid=(nsteps,),
        in_specs=[
            pl.BlockSpec((EBLK, DE), lambda i: (i, 0)),
            pl.BlockSpec((DE, D), lambda i: (0, 0)),
            pl.BlockSpec((1, D), lambda i: (0, 0)),
        ],
        out_specs=pl.BlockSpec((EBLK, D), lambda i: (i, 0)),
        out_shape=jax.ShapeDtypeStruct((EPAD, D), jnp.float32),
        compiler_params=pltpu.CompilerParams(
            dimension_semantics=("arbitrary",)),
    )(ea_pad, We_l, be_l)


MBLK = 1024


def _mlp_body(h_ref, a0_ref, a1_ref, ep_ref, w1_ref, b1_ref, w2_ref, b2_ref,
              o_ref):
    t = h_ref[...] * ep_ref[...] + a0_ref[...] + a1_ref[...]
    z = jnp.maximum(
        jnp.dot(t, w1_ref[...], preferred_element_type=jnp.float32)
        + b1_ref[...], 0.0)
    o_ref[...] = jnp.maximum(
        jnp.dot(z, w2_ref[...], preferred_element_type=jnp.float32)
        + b2_ref[...], 0.0)


def _mlp(h, a0, a1, epsv, W1l, b1l, W2l, b2l):
    nsteps = NPAD // MBLK
    return pl.pallas_call(
        _mlp_body,
        grid=(nsteps,),
        in_specs=[
            pl.BlockSpec((MBLK, D), lambda i: (i, 0)),
            pl.BlockSpec((MBLK, D), lambda i: (i, 0)),
            pl.BlockSpec((MBLK, D), lambda i: (i, 0)),
            pl.BlockSpec((1, D), lambda i: (0, 0)),
            pl.BlockSpec((D, D), lambda i: (0, 0)),
            pl.BlockSpec((1, D), lambda i: (0, 0)),
            pl.BlockSpec((D, D), lambda i: (0, 0)),
            pl.BlockSpec((1, D), lambda i: (0, 0)),
        ],
        out_specs=pl.BlockSpec((MBLK, D), lambda i: (i, 0)),
        out_shape=jax.ShapeDtypeStruct((NPAD, D), jnp.float32),
        compiler_params=pltpu.CompilerParams(
            dimension_semantics=("arbitrary",)),
    )(h, a0, a1, epsv, W1l, b1l, W2l, b2l)


PBLK = 1024


def _pool_body(h_ref, b_ref, o_ref, acc_s, acc_c):
    i = pl.program_id(0)

    @pl.when(i == 0)
    def _init():
        acc_s[...] = jnp.zeros_like(acc_s)
        acc_c[...] = jnp.zeros_like(acc_c)

    bt = b_ref[...].reshape(1, PBLK)
    gi = lax.broadcasted_iota(jnp.int32, (G, PBLK), 0)
    oh = (gi == bt).astype(jnp.float32)
    acc_s[...] += jnp.dot(oh, h_ref[...], preferred_element_type=jnp.float32)
    acc_c[...] = acc_c[...] + jnp.sum(oh, axis=1, keepdims=True)

    @pl.when(i == NPAD // PBLK - 1)
    def _fin():
        o_ref[...] = acc_s[...] / jnp.maximum(acc_c[...], 1.0)


def _pool(h, batch2d):
    nsteps = NPAD // PBLK
    return pl.pallas_call(
        _pool_body,
        grid=(nsteps,),
        in_specs=[
            pl.BlockSpec((PBLK, D), lambda i: (i, 0)),
            pl.BlockSpec((PBLK, 1), lambda i: (i, 0)),
        ],
        out_specs=pl.BlockSpec((G, D), lambda i: (0, 0)),
        out_shape=jax.ShapeDtypeStruct((G, D), jnp.float32),
        scratch_shapes=[
            pltpu.VMEM((G, D), jnp.float32),
            pltpu.VMEM((G, 128), jnp.float32),
        ],
        compiler_params=pltpu.CompilerParams(
            dimension_semantics=("arbitrary",)),
    )(h, batch2d)


def _head_body(hg_ref, wp_ref, bp_ref, o_ref):
    o_ref[0] = (jnp.dot(hg_ref[...], wp_ref[0],
                        preferred_element_type=jnp.float32)
                + bp_ref[0])


def _heads(hg, Wp_pad, bp_pad):
    return pl.pallas_call(
        _head_body,
        grid=(S, VPAD // VBLK),
        in_specs=[
            pl.BlockSpec((G, D), lambda s, v: (0, 0)),
            pl.BlockSpec((1, D, VBLK), lambda s, v: (s, 0, v)),
            pl.BlockSpec((1, 1, VBLK), lambda s, v: (s, 0, v)),
        ],
        out_specs=pl.BlockSpec((1, G, VBLK), lambda s, v: (s, 0, v)),
        out_shape=jax.ShapeDtypeStruct((S, G, VPAD), jnp.float32),
        compiler_params=pltpu.CompilerParams(
            dimension_semantics=("arbitrary", "arbitrary")),
    )(hg, Wp_pad, bp_pad)


def kernel(x, node_depth, edge_index, edge_attr, batch,
           emb_type, emb_attr, emb_depth,
           We, be, W1, b1, W2, b2, eps, Wp, bp):
    f32 = jnp.float32
    i32 = jnp.int32

    x0r = jnp.pad(x[:, 0].astype(i32), (0, NPAD - N)).reshape(NROWS, 128)
    x1r = jnp.pad(x[:, 1].astype(i32), (0, NPAD - N)).reshape(NROWS, 128)
    ddr = jnp.pad(node_depth[:, 0].astype(i32),
                  (0, NPAD - N)).reshape(NROWS, 128)
    src2d = jnp.pad(
        jnp.pad(edge_index[0].astype(i32), (0, EPAD - E)).reshape(ECHUNKS, CE),
        ((0, 8), (0, 0)))
    dst2d = jnp.pad(
        jnp.pad(edge_index[1].astype(i32), (0, EPAD - E),
                constant_values=N).reshape(ECHUNKS, CE),
        ((0, 8), (0, 0)), constant_values=N)
    ea_pad = jnp.pad(edge_attr.astype(f32), ((0, EPAD - E), (0, 0)))
    batch2d = jnp.pad(batch.astype(i32), (0, NPAD - N),
                      constant_values=G).reshape(NPAD, 1)

    h = _node_encoder(x0r, x1r, ddr, emb_type.astype(f32),
                      emb_attr.astype(f32), emb_depth.astype(f32))

    for l in range(NLAYER):
        e_l = _edge_encoder(ea_pad, We[l].astype(f32),
                            be[l][None, :].astype(f32))
        aggp = _edge_aggregate(src2d, dst2d, e_l, h)
        epsv = (1.0 + eps[l]) * jnp.ones((1, D), f32)
        h = _mlp(h, aggp[0], aggp[1], epsv,
                 W1[l].astype(f32), b1[l][None, :].astype(f32),
                 W2[l].astype(f32), b2[l][None, :].astype(f32))

    hg = _pool(h, batch2d)
    Wp_pad = jnp.pad(Wp.astype(f32), ((0, 0), (0, 0), (0, VPAD - V)))
    bp_pad = jnp.pad(bp.astype(f32), ((0, 0), (0, VPAD - V))).reshape(S, 1, VPAD)
    preds = _heads(hg, Wp_pad, bp_pad)
    return preds[:, :, :V]
